# Initial kernel scaffold; baseline (speedup 1.0000x reference)
import jax
import jax.numpy as jnp
from jax.experimental import pallas as pl


def kernel(x, edge_index, W1, att_src1, att_dst1, bias1, gamma, beta, W2, att_src2, att_dst2, bias2):
    raise NotImplementedError("write your pallas kernel here")


# speedup vs baseline: 1.0136x; 1.0136x over previous
import functools

import jax
import jax.numpy as jnp
from jax.experimental import pallas as pl

_N = 10000
_E = 320000
_D = 128
_H = 8
_HID = 128


def _mm_body(x_ref, w_ref, o_ref):
    o_ref[...] = jnp.dot(x_ref[...], w_ref[...],
                         preferred_element_type=jnp.float32)


def _matmul(x, w, blk_m=2000):
    m, k = x.shape
    _, n = w.shape
    return pl.pallas_call(
        _mm_body,
        grid=(m // blk_m,),
        in_specs=[
            pl.BlockSpec((blk_m, k), lambda i: (i, 0)),
            pl.BlockSpec((k, n), lambda i: (0, 0)),
        ],
        out_specs=pl.BlockSpec((blk_m, n), lambda i: (i, 0)),
        out_shape=jax.ShapeDtypeStruct((m, n), jnp.float32),
    )(x, w)


def _gat_layer(h_nodes, a_s, a_d, src, dst, heads, out_ch):
    n = h_nodes.shape[0]
    h = h_nodes.reshape(n, heads, out_ch)
    loop = jnp.arange(n, dtype=src.dtype)
    src = jnp.concatenate([src, loop])
    dst = jnp.concatenate([dst, loop])
    e = jax.nn.leaky_relu(a_s[src] + a_d[dst], negative_slope=0.2)
    m = jax.ops.segment_max(e, dst, num_segments=n)
    ex = jnp.exp(e - m[dst])
    s = jax.ops.segment_sum(ex, dst, num_segments=n)
    alpha = ex / (s[dst] + 1e-16)
    out = jax.ops.segment_sum(h[src] * alpha[:, :, None], dst, num_segments=n)
    return out


def kernel(x, edge_index, W1, att_src1, att_dst1, bias1, gamma, beta, W2,
           att_src2, att_dst2, bias2):
    src, dst = edge_index[0], edge_index[1]

    W1r = W1.reshape(_D, _H, _HID)
    As1 = jnp.einsum('dhc,hc->dh', W1r, att_src1)
    Ad1 = jnp.einsum('dhc,hc->dh', W1r, att_dst1)
    pad1 = jnp.zeros((_D, 128 - 2 * _H), jnp.float32)
    W1e = jnp.concatenate([W1, As1, Ad1, pad1], axis=1)

    h1e = _matmul(x, W1e)
    h1 = h1e[:, :_H * _HID]
    a_s1 = h1e[:, _H * _HID:_H * _HID + _H]
    a_d1 = h1e[:, _H * _HID + _H:_H * _HID + 2 * _H]

    out1 = _gat_layer(h1, a_s1, a_d1, src, dst, _H, _HID)
    h = out1.reshape(_N, _H * _HID) + bias1

    mean = jnp.mean(h, axis=0)
    var = jnp.var(h, axis=0)
    h = (h - mean) / jnp.sqrt(var + 1e-5) * gamma + beta
    h = jax.nn.elu(h)

    W2r = W2.reshape(_H * _HID, 1, _D)
    As2 = jnp.einsum('dhc,hc->dh', W2r, att_src2)
    Ad2 = jnp.einsum('dhc,hc->dh', W2r, att_dst2)
    pad2 = jnp.zeros((_H * _HID, 128 - 2), jnp.float32)
    W2e = jnp.concatenate([W2, As2, Ad2, pad2], axis=1)

    h2e = _matmul(h, W2e)
    z2 = h2e[:, :_D]
    a_s2 = h2e[:, _D:_D + 1]
    a_d2 = h2e[:, _D + 1:_D + 2]

    out2 = _gat_layer(z2, a_s2, a_d2, src, dst, 1, _D)
    return out2[:, 0, :] + bias2


# speedup vs baseline: 1.2241x; 1.2076x over previous
import functools

import jax
import jax.numpy as jnp
from jax import lax
from jax.experimental import pallas as pl
from jax.experimental.pallas import tpu as pltpu
from jax.experimental.pallas import tpu_sc as plsc

_N = 10000
_E = 320000
_D = 128
_H = 8
_HID = 128

_NTILES = 32
_EPT = _E // _NTILES
_B = 80
_NCHUNK = _EPT // _B
_NPAD = 10240
_RPT = _NPAD // 16
_RB = 80
_W2COLS = 144


def _mm_body(x_ref, w_ref, o_ref):
    o_ref[...] = jnp.dot(x_ref[...], w_ref[...],
                         preferred_element_type=jnp.float32)


def _matmul(x, w, blk_m=2000):
    m, k = x.shape
    _, n = w.shape
    return pl.pallas_call(
        _mm_body,
        grid=(m // blk_m,),
        in_specs=[
            pl.BlockSpec((blk_m, k), lambda i: (i, 0)),
            pl.BlockSpec((k, n), lambda i: (0, 0)),
        ],
        out_specs=pl.BlockSpec((blk_m, n), lambda i: (i, 0)),
        out_shape=jax.ShapeDtypeStruct((m, n), jnp.float32),
    )(x, w)


def _gat_layer_xla(h_nodes, a_s, a_d, src, dst, heads, out_ch):
    n = h_nodes.shape[0]
    h = h_nodes.reshape(n, heads, out_ch)
    loop = jnp.arange(n, dtype=src.dtype)
    src = jnp.concatenate([src, loop])
    dst = jnp.concatenate([dst, loop])
    e = jax.nn.leaky_relu(a_s[src] + a_d[dst], negative_slope=0.2)
    m = jax.ops.segment_max(e, dst, num_segments=n)
    ex = jnp.exp(e - m[dst])
    s = jax.ops.segment_sum(ex, dst, num_segments=n)
    alpha = ex / (s[dst] + 1e-16)
    out = jax.ops.segment_sum(h[src] * alpha[:, :, None], dst, num_segments=n)
    return out


def _sc_gat2_body(src_h, dst_h, z2e_h, as2_h, ad2_h, m2_h, zr_h,
                  out_h,
                  as2v, ad2v, srcb, dstb, exb, rows, mv, acc, sem):
    cid = lax.axis_index("c")
    sid = lax.axis_index("s")
    wid = sid * 2 + cid

    pltpu.sync_copy(as2_h, as2v)
    pltpu.sync_copy(ad2_h, ad2v)
    pltpu.sync_copy(m2_h, mv)
    pltpu.sync_copy(zr_h, rows)
    for i in range(_RPT // _RB):
        pltpu.sync_copy(rows, acc.at[pl.ds(sid * _RPT + i * _RB, _RB)])
    plsc.subcore_barrier()

    def chunk(c, carry):
        base = wid * _EPT + c * _B
        pltpu.sync_copy(src_h.at[pl.ds(base, _B)], srcb)
        pltpu.sync_copy(dst_h.at[pl.ds(base, _B)], dstb)
        mvv = mv[...]
        for k in range(_B // 16):
            iv = srcb[pl.ds(k * 16, 16)]
            jv = dstb[pl.ds(k * 16, 16)]
            vs = plsc.load_gather(as2v, [iv])
            vd = plsc.load_gather(ad2v, [jv])
            e = vs + vd
            e = jnp.where(e >= 0, e, 0.2 * e)
            exb[pl.ds(k * 16, 16)] = jnp.exp(e - mvv)
        pltpu.async_copy(z2e_h.at[srcb], rows, sem).wait()

        def jbody(j, carry2):
            w = plsc.load_gather(exb, [jnp.zeros((16,), jnp.int32) + j])
            for k in range(_W2COLS // 16):
                sl = pl.ds(k * 16, 16)
                rows[j, sl] = rows[j, sl] * w
            return carry2

        lax.fori_loop(0, _B, jbody, 0)
        pltpu.sync_copy(rows, acc.at[dstb], add=True)
        return carry

    lax.fori_loop(0, _NCHUNK, chunk, 0)
    plsc.subcore_barrier()

    for i in range(_RPT // _RB):
        sl = pl.ds(sid * _RPT + i * _RB, _RB)
        pltpu.sync_copy(acc.at[sl], rows)
        pltpu.sync_copy(rows, out_h.at[cid, sl])


def _sc_gat2(src, dst, z2e, as2, ad2, m2vec):
    zr = jnp.zeros((_RB, _W2COLS), jnp.float32)
    mesh = plsc.VectorSubcoreMesh(core_axis_name="c", subcore_axis_name="s")
    f = pl.kernel(
        _sc_gat2_body,
        out_type=jax.ShapeDtypeStruct((2, _NPAD, _W2COLS), jnp.float32),
        mesh=mesh,
        scratch_types=[
            pltpu.VMEM((_N,), jnp.float32),
            pltpu.VMEM((_N,), jnp.float32),
            pltpu.VMEM((_B,), jnp.int32),
            pltpu.VMEM((_B,), jnp.int32),
            pltpu.VMEM((_B,), jnp.float32),
            pltpu.VMEM((_B, _W2COLS), jnp.float32),
            pltpu.VMEM((16,), jnp.float32),
            pltpu.VMEM_SHARED((_NPAD, _W2COLS), jnp.float32),
            pltpu.SemaphoreType.DMA,
        ],
        compiler_params=pltpu.CompilerParams(
            use_tc_tiling_on_sc=False, needs_layout_passes=False),
    )
    return f(src, dst, z2e, as2, ad2, m2vec, zr)


def _k4_body(p_ref, z_ref, aux_ref, m2_ref, b2_ref, o_ref):
    p0 = p_ref[0]
    p1 = p_ref[1]
    z = z_ref[...]
    a = aux_ref[...]
    m2 = m2_ref[...][0:1, 0:1]
    el = a[:, 0:1] + a[:, 1:2]
    el = jnp.where(el >= 0, el, 0.2 * el)
    exw = jnp.exp(el - m2)
    num = p0[:, :_D] + p1[:, :_D] + exw * z[:, :_D]
    den = p0[:, _D:_D + 1] + p1[:, _D:_D + 1] + exw
    o_ref[...] = num / den + b2_ref[...][0:1, :]


def _k4(out2p, z2e, aux2, m2arr, bias2b, blk=1000):
    return pl.pallas_call(
        _k4_body,
        grid=(_N // blk,),
        in_specs=[
            pl.BlockSpec((2, blk, _W2COLS), lambda i: (0, i, 0)),
            pl.BlockSpec((blk, _W2COLS), lambda i: (i, 0)),
            pl.BlockSpec((blk, 16), lambda i: (i, 0)),
            pl.BlockSpec((8, 128), lambda i: (0, 0)),
            pl.BlockSpec((8, 128), lambda i: (0, 0)),
        ],
        out_specs=pl.BlockSpec((blk, _D), lambda i: (i, 0)),
        out_shape=jax.ShapeDtypeStruct((_N, _D), jnp.float32),
    )(out2p, z2e, aux2, m2arr, bias2b)


def kernel(x, edge_index, W1, att_src1, att_dst1, bias1, gamma, beta, W2,
           att_src2, att_dst2, bias2):
    src, dst = edge_index[0], edge_index[1]

    W1r = W1.reshape(_D, _H, _HID)
    As1 = jnp.einsum('dhc,hc->dh', W1r, att_src1)
    Ad1 = jnp.einsum('dhc,hc->dh', W1r, att_dst1)
    pad1 = jnp.zeros((_D, 128 - 2 * _H), jnp.float32)
    W1e = jnp.concatenate([W1, As1, Ad1, pad1], axis=1)

    h1e = _matmul(x, W1e)
    h1 = h1e[:, :_H * _HID]
    a_s1 = h1e[:, _H * _HID:_H * _HID + _H]
    a_d1 = h1e[:, _H * _HID + _H:_H * _HID + 2 * _H]

    out1 = _gat_layer_xla(h1, a_s1, a_d1, src, dst, _H, _HID)
    h = out1.reshape(_N, _H * _HID) + bias1

    mean = jnp.mean(h, axis=0)
    var = jnp.var(h, axis=0)
    h = (h - mean) / jnp.sqrt(var + 1e-5) * gamma + beta
    h = jax.nn.elu(h)

    W2r = W2.reshape(_H * _HID, 1, _D)
    As2 = jnp.einsum('dhc,hc->dh', W2r, att_src2)
    Ad2 = jnp.einsum('dhc,hc->dh', W2r, att_dst2)
    pad2 = jnp.zeros((_H * _HID, 14), jnp.float32)
    W2e = jnp.concatenate([W2, As2, Ad2, pad2], axis=1)

    h2e = _matmul(h, W2e)
    z2 = h2e[:, :_D]
    as2 = h2e[:, _D]
    ad2 = h2e[:, _D + 1]
    aux2 = h2e[:, _D:_D + 16]

    m2 = jnp.max(as2) + jnp.max(ad2)
    m2vec = jnp.full((16,), m2, jnp.float32)
    m2arr = jnp.full((8, 128), m2, jnp.float32)

    onescol = (lax.broadcasted_iota(jnp.int32, (_N, _W2COLS - _D), 1)
               == 0).astype(jnp.float32)
    z2e = jnp.concatenate([z2, onescol], axis=1)

    out2p = _sc_gat2(src, dst, z2e, as2, ad2, m2vec)[:, :_N, :]

    bias2b = jnp.broadcast_to(bias2[None, :], (8, 128))
    return _k4(out2p, z2e, aux2, m2arr, bias2b)


# speedup vs baseline: 10.9257x; 8.9257x over previous
import functools

import jax
import jax.numpy as jnp
from jax import lax
from jax.experimental import pallas as pl
from jax.experimental.pallas import tpu as pltpu
from jax.experimental.pallas import tpu_sc as plsc

_N = 10000
_E = 320000
_D = 128
_H = 8
_HID = 128

_NTILES = 32
_EPT = _E // _NTILES
_B = 80
_NCHUNK = _EPT // _B
_NPAD = 10240
_RPT = _NPAD // 16
_W2COLS = 144

_SC_PARAMS = pltpu.CompilerParams(
    use_tc_tiling_on_sc=False, needs_layout_passes=False)


def _mesh():
    return plsc.VectorSubcoreMesh(core_axis_name="c", subcore_axis_name="s")


def _mm2_body(x_ref, w1_ref, w2_ref, o1_ref, o2_ref):
    x = x_ref[...]
    o1_ref[...] = jnp.dot(x, w1_ref[...], preferred_element_type=jnp.float32)
    o2_ref[...] = jnp.dot(x, w2_ref[...], preferred_element_type=jnp.float32)


def _matmul2(x, w1, w2, blk_m=2000):
    m, k = x.shape
    return pl.pallas_call(
        _mm2_body,
        grid=(m // blk_m,),
        in_specs=[
            pl.BlockSpec((blk_m, k), lambda i: (i, 0)),
            pl.BlockSpec((k, w1.shape[1]), lambda i: (0, 0)),
            pl.BlockSpec((k, w2.shape[1]), lambda i: (0, 0)),
        ],
        out_specs=[
            pl.BlockSpec((blk_m, w1.shape[1]), lambda i: (i, 0)),
            pl.BlockSpec((blk_m, w2.shape[1]), lambda i: (i, 0)),
        ],
        out_shape=[
            jax.ShapeDtypeStruct((m, w1.shape[1]), jnp.float32),
            jax.ShapeDtypeStruct((m, w2.shape[1]), jnp.float32),
        ],
    )(x, w1, w2)


def _s1_body(src_h, dst_h, asd_h, m_h, zr_h,
             ext_h, s1p_h,
             srcb, dstb, rs, rd, exs, exT, mv, acc, sem):
    cid = lax.axis_index("c")
    sid = lax.axis_index("s")
    wid = sid * 2 + cid
    perm = (lax.iota(jnp.int32, 16) % 8) + 8

    pltpu.sync_copy(m_h, mv)
    pltpu.sync_copy(zr_h, exs)
    for i in range(_RPT // _B):
        pltpu.sync_copy(exs, acc.at[pl.ds(sid * _RPT + i * _B, _B)])
    plsc.subcore_barrier()

    def chunk(c, carry):
        base = wid * _EPT + c * _B
        pltpu.sync_copy(src_h.at[pl.ds(base, _B)], srcb)
        pltpu.sync_copy(dst_h.at[pl.ds(base, _B)], dstb)
        pltpu.async_copy(asd_h.at[srcb], rs, sem).wait()
        pltpu.async_copy(asd_h.at[dstb], rd, sem).wait()
        mvv = mv[...]

        def jbody(j, carry2):
            a = rs[j, :]
            bp = plsc.load_gather(rd, [jnp.zeros((16,), jnp.int32) + j, perm])
            e = a + bp
            e = jnp.where(e >= 0, e, 0.2 * e)
            ex = jnp.exp(e - mvv)
            exs[j, :] = ex
            plsc.store_scatter(
                exT, [lax.iota(jnp.int32, 16),
                      jnp.zeros((16,), jnp.int32) + (c * _B + j)],
                ex, mask=lax.iota(jnp.int32, 16) < 8)
            return carry2

        lax.fori_loop(0, _B, jbody, 0)
        pltpu.sync_copy(exs, acc.at[dstb], add=True)
        return carry

    lax.fori_loop(0, _NCHUNK, chunk, 0)
    plsc.subcore_barrier()

    pltpu.sync_copy(exT, ext_h.at[wid])
    for i in range(_RPT // _B):
        sl = pl.ds(sid * _RPT + i * _B, _B)
        pltpu.sync_copy(acc.at[sl], rs)
        pltpu.sync_copy(rs, s1p_h.at[cid, sl])


def _s1(src, dst, asd, m1vec):
    zr = jnp.zeros((_B, 16), jnp.float32)
    f = pl.kernel(
        _s1_body,
        out_type=[
            jax.ShapeDtypeStruct((_NTILES, _H, _EPT), jnp.float32),
            jax.ShapeDtypeStruct((2, _NPAD, 16), jnp.float32),
        ],
        mesh=_mesh(),
        scratch_types=[
            pltpu.VMEM((_B,), jnp.int32),
            pltpu.VMEM((_B,), jnp.int32),
            pltpu.VMEM((_B, 16), jnp.float32),
            pltpu.VMEM((_B, 16), jnp.float32),
            pltpu.VMEM((_B, 16), jnp.float32),
            pltpu.VMEM((_H, _EPT), jnp.float32),
            pltpu.VMEM((16,), jnp.float32),
            pltpu.VMEM_SHARED((_NPAD, 16), jnp.float32),
            pltpu.SemaphoreType.DMA,
        ],
        compiler_params=_SC_PARAMS,
    )
    return f(src, dst, asd, m1vec, zr)


def _s2_body(src_h, dst_h, h1r_h, ext_h, zr_h,
             out_h,
             srcb, dstb, gidx, exb, rows, acc, sem):
    cid = lax.axis_index("c")
    sid = lax.axis_index("s")
    wid = sid * 2 + cid

    for h in range(_H):
        pltpu.sync_copy(zr_h, rows)
        for i in range(_RPT // _B):
            pltpu.sync_copy(rows, acc.at[pl.ds(sid * _RPT + i * _B, _B)])
        plsc.subcore_barrier()

        def chunk(c, carry):
            base = wid * _EPT + c * _B
            pltpu.sync_copy(src_h.at[pl.ds(base, _B)], srcb)
            pltpu.sync_copy(dst_h.at[pl.ds(base, _B)], dstb)
            pltpu.sync_copy(ext_h.at[wid, h, pl.ds(c * _B, _B)], exb)
            for k in range(_B // 16):
                sl = pl.ds(k * 16, 16)
                gidx[sl] = srcb[sl] * 8 + h
            pltpu.async_copy(h1r_h.at[gidx], rows, sem).wait()

            def jbody(j, carry2):
                w = plsc.load_gather(exb, [jnp.zeros((16,), jnp.int32) + j])
                for k in range(_D // 16):
                    sl = pl.ds(k * 16, 16)
                    rows[j, sl] = rows[j, sl] * w
                return carry2

            lax.fori_loop(0, _B, jbody, 0)
            pltpu.sync_copy(rows, acc.at[dstb], add=True)
            return carry

        lax.fori_loop(0, _NCHUNK, chunk, 0)
        plsc.subcore_barrier()

        for i in range(_RPT // _B):
            sl = pl.ds(sid * _RPT + i * _B, _B)
            pltpu.sync_copy(acc.at[sl], rows)
            pltpu.sync_copy(rows, out_h.at[cid, h, sl])
        plsc.subcore_barrier()


def _s2(src, dst, h1r, ext):
    zr = jnp.zeros((_B, _D), jnp.float32)
    f = pl.kernel(
        _s2_body,
        out_type=jax.ShapeDtypeStruct((2, _H, _NPAD, _D), jnp.float32),
        mesh=_mesh(),
        scratch_types=[
            pltpu.VMEM((_B,), jnp.int32),
            pltpu.VMEM((_B,), jnp.int32),
            pltpu.VMEM((_B,), jnp.int32),
            pltpu.VMEM((_B,), jnp.float32),
            pltpu.VMEM((_B, _D), jnp.float32),
            pltpu.VMEM_SHARED((_NPAD, _D), jnp.float32),
            pltpu.SemaphoreType.DMA,
        ],
        compiler_params=_SC_PARAMS,
    )
    return f(src, dst, h1r, ext, zr)


def _k3a_body(p_ref, h1_ref, aux_ref, s_ref, m1_ref, b1_ref,
              o_ref, sums_ref, sq_ref):
    aux = aux_ref[...]
    a_s = aux[:, :_H]
    a_d = aux[:, _H:2 * _H]
    el = a_s + a_d
    el = jnp.where(el >= 0, el, 0.2 * el)
    exw = jnp.exp(el - m1_ref[...][0:1, :])
    s_tot = s_ref[0][:, :_H] + s_ref[1][:, :_H] + exw
    inv = 1.0 / s_tot
    parts = []
    for h in range(_H):
        num = (p_ref[0, h] + p_ref[1, h]
               + exw[:, h:h + 1] * h1_ref[:, h * _HID:(h + 1) * _HID])
        parts.append(num * inv[:, h:h + 1])
    hout = jnp.concatenate(parts, axis=1) + b1_ref[...][0:1, :]
    o_ref[...] = hout
    cs = jnp.broadcast_to(jnp.sum(hout, axis=0)[None, :], (8, _H * _HID))
    css = jnp.broadcast_to(jnp.sum(hout * hout, axis=0)[None, :],
                           (8, _H * _HID))

    @pl.when(pl.program_id(0) == 0)
    def _():
        sums_ref[...] = cs
        sq_ref[...] = css

    @pl.when(pl.program_id(0) > 0)
    def _():
        sums_ref[...] += cs
        sq_ref[...] += css


def _k3a(out1p, h1, aux, s1p, m1b, bias1b, blk=1000):
    return pl.pallas_call(
        _k3a_body,
        grid=(_N // blk,),
        in_specs=[
            pl.BlockSpec((2, _H, blk, _D), lambda i: (0, 0, i, 0)),
            pl.BlockSpec((blk, _H * _HID), lambda i: (i, 0)),
            pl.BlockSpec((blk, _D), lambda i: (i, 0)),
            pl.BlockSpec((2, blk, 16), lambda i: (0, i, 0)),
            pl.BlockSpec((8, _H), lambda i: (0, 0)),
            pl.BlockSpec((8, _H * _HID), lambda i: (0, 0)),
        ],
        out_specs=[
            pl.BlockSpec((blk, _H * _HID), lambda i: (i, 0)),
            pl.BlockSpec((8, _H * _HID), lambda i: (0, 0)),
            pl.BlockSpec((8, _H * _HID), lambda i: (0, 0)),
        ],
        out_shape=[
            jax.ShapeDtypeStruct((_N, _H * _HID), jnp.float32),
            jax.ShapeDtypeStruct((8, _H * _HID), jnp.float32),
            jax.ShapeDtypeStruct((8, _H * _HID), jnp.float32),
        ],
    )(out1p, h1, aux, s1p, m1b, bias1b)


def _k3b_body(h_ref, sc_ref, sh_ref, w_ref, o_ref):
    hb = h_ref[...] * sc_ref[...][0:1, :] + sh_ref[...][0:1, :]
    he = jnp.where(hb > 0, hb, jnp.exp(hb) - 1.0)
    o_ref[...] = jnp.dot(he, w_ref[...], preferred_element_type=jnp.float32)


def _k3b(h, scale8, shift8, w2e, blk=1000):
    k = h.shape[1]
    n = w2e.shape[1]
    return pl.pallas_call(
        _k3b_body,
        grid=(_N // blk,),
        in_specs=[
            pl.BlockSpec((blk, k), lambda i: (i, 0)),
            pl.BlockSpec((8, k), lambda i: (0, 0)),
            pl.BlockSpec((8, k), lambda i: (0, 0)),
            pl.BlockSpec((k, n), lambda i: (0, 0)),
        ],
        out_specs=pl.BlockSpec((blk, n), lambda i: (i, 0)),
        out_shape=jax.ShapeDtypeStruct((_N, n), jnp.float32),
    )(h, scale8, shift8, w2e)


def _s3_body(src_h, dst_h, z2e_h, as2_h, ad2_h, m2_h, zr_h,
             out_h,
             as2v, ad2v, srcb, dstb, exb, rows, mv, acc, sem):
    cid = lax.axis_index("c")
    sid = lax.axis_index("s")
    wid = sid * 2 + cid

    pltpu.sync_copy(as2_h, as2v)
    pltpu.sync_copy(ad2_h, ad2v)
    pltpu.sync_copy(m2_h, mv)
    pltpu.sync_copy(zr_h, rows)
    for i in range(_RPT // _B):
        pltpu.sync_copy(rows, acc.at[pl.ds(sid * _RPT + i * _B, _B)])
    plsc.subcore_barrier()

    def chunk(c, carry):
        base = wid * _EPT + c * _B
        pltpu.sync_copy(src_h.at[pl.ds(base, _B)], srcb)
        pltpu.sync_copy(dst_h.at[pl.ds(base, _B)], dstb)
        mvv = mv[...]
        for k in range(_B // 16):
            sl = pl.ds(k * 16, 16)
            vs = plsc.load_gather(as2v, [srcb[sl]])
            vd = plsc.load_gather(ad2v, [dstb[sl]])
            e = vs + vd
            e = jnp.where(e >= 0, e, 0.2 * e)
            exb[sl] = jnp.exp(e - mvv)
        pltpu.async_copy(z2e_h.at[srcb], rows, sem).wait()

        def jbody(j, carry2):
            w = plsc.load_gather(exb, [jnp.zeros((16,), jnp.int32) + j])
            for k in range(_W2COLS // 16):
                sl = pl.ds(k * 16, 16)
                rows[j, sl] = rows[j, sl] * w
            return carry2

        lax.fori_loop(0, _B, jbody, 0)
        pltpu.sync_copy(rows, acc.at[dstb], add=True)
        return carry

    lax.fori_loop(0, _NCHUNK, chunk, 0)
    plsc.subcore_barrier()

    for i in range(_RPT // _B):
        sl = pl.ds(sid * _RPT + i * _B, _B)
        pltpu.sync_copy(acc.at[sl], rows)
        pltpu.sync_copy(rows, out_h.at[cid, sl])


def _s3(src, dst, z2e, as2, ad2, m2vec):
    zr = jnp.zeros((_B, _W2COLS), jnp.float32)
    f = pl.kernel(
        _s3_body,
        out_type=jax.ShapeDtypeStruct((2, _NPAD, _W2COLS), jnp.float32),
        mesh=_mesh(),
        scratch_types=[
            pltpu.VMEM((_N,), jnp.float32),
            pltpu.VMEM((_N,), jnp.float32),
            pltpu.VMEM((_B,), jnp.int32),
            pltpu.VMEM((_B,), jnp.int32),
            pltpu.VMEM((_B,), jnp.float32),
            pltpu.VMEM((_B, _W2COLS), jnp.float32),
            pltpu.VMEM((16,), jnp.float32),
            pltpu.VMEM_SHARED((_NPAD, _W2COLS), jnp.float32),
            pltpu.SemaphoreType.DMA,
        ],
        compiler_params=_SC_PARAMS,
    )
    return f(src, dst, z2e, as2, ad2, m2vec, zr)


def _k4_body(p_ref, z_ref, aux_ref, m2_ref, b2_ref, o_ref):
    p0 = p_ref[0]
    p1 = p_ref[1]
    z = z_ref[...]
    a = aux_ref[...]
    m2 = m2_ref[...][0:1, 0:1]
    el = a[:, 0:1] + a[:, 1:2]
    el = jnp.where(el >= 0, el, 0.2 * el)
    exw = jnp.exp(el - m2)
    num = p0[:, :_D] + p1[:, :_D] + exw * z[:, :_D]
    den = p0[:, _D:_D + 1] + p1[:, _D:_D + 1] + exw
    o_ref[...] = num / den + b2_ref[...][0:1, :]


def _k4(out2p, z2e, aux2, m2arr, bias2b, blk=1000):
    return pl.pallas_call(
        _k4_body,
        grid=(_N // blk,),
        in_specs=[
            pl.BlockSpec((2, blk, _W2COLS), lambda i: (0, i, 0)),
            pl.BlockSpec((blk, _W2COLS), lambda i: (i, 0)),
            pl.BlockSpec((blk, 16), lambda i: (i, 0)),
            pl.BlockSpec((8, 128), lambda i: (0, 0)),
            pl.BlockSpec((8, 128), lambda i: (0, 0)),
        ],
        out_specs=pl.BlockSpec((blk, _D), lambda i: (i, 0)),
        out_shape=jax.ShapeDtypeStruct((_N, _D), jnp.float32),
    )(out2p, z2e, aux2, m2arr, bias2b)


def kernel(x, edge_index, W1, att_src1, att_dst1, bias1, gamma, beta, W2,
           att_src2, att_dst2, bias2):
    src, dst = edge_index[0], edge_index[1]

    W1r = W1.reshape(_D, _H, _HID)
    As1 = jnp.einsum('dhc,hc->dh', W1r, att_src1)
    Ad1 = jnp.einsum('dhc,hc->dh', W1r, att_dst1)
    pad1 = jnp.zeros((_D, 128 - 2 * _H), jnp.float32)
    Waux1 = jnp.concatenate([As1, Ad1, pad1], axis=1)

    h1, aux = _matmul2(x, W1, Waux1)
    asd = aux[:, :16]
    m1 = jnp.max(aux[:, :_H], axis=0) + jnp.max(aux[:, _H:2 * _H], axis=0)
    m1vec = jnp.concatenate([m1, jnp.full((8,), 1e30, jnp.float32)])
    m1b = jnp.broadcast_to(m1[None, :], (8, _H))

    ext, s1p = _s1(src, dst, asd, m1vec)
    h1r = h1.reshape(_N * _H, _HID)
    out1p = _s2(src, dst, h1r, ext)

    bias1b = jnp.broadcast_to(bias1[None, :], (8, _H * _HID))
    h, sums, sq = _k3a(out1p[:, :, :_N, :], h1, aux, s1p[:, :_N, :],
                       m1b, bias1b)

    mean = sums[0] / _N
    var = sq[0] / _N - mean * mean
    scale = gamma / jnp.sqrt(var + 1e-5)
    shift = beta - mean * scale
    scale8 = jnp.broadcast_to(scale[None, :], (8, _H * _HID))
    shift8 = jnp.broadcast_to(shift[None, :], (8, _H * _HID))

    W2r = W2.reshape(_H * _HID, 1, _D)
    As2 = jnp.einsum('dhc,hc->dh', W2r, att_src2)
    Ad2 = jnp.einsum('dhc,hc->dh', W2r, att_dst2)
    pad2 = jnp.zeros((_H * _HID, 14), jnp.float32)
    W2e = jnp.concatenate([W2, As2, Ad2, pad2], axis=1)

    h2e = _k3b(h, scale8, shift8, W2e)
    z2 = h2e[:, :_D]
    as2 = h2e[:, _D]
    ad2 = h2e[:, _D + 1]
    aux2 = h2e[:, _D:_D + 16]

    m2 = jnp.max(as2) + jnp.max(ad2)
    m2vec = jnp.full((16,), m2, jnp.float32)
    m2arr = jnp.full((8, 128), m2, jnp.float32)

    onescol = (lax.broadcasted_iota(jnp.int32, (_N, _W2COLS - _D), 1)
               == 0).astype(jnp.float32)
    z2e = jnp.concatenate([z2, onescol], axis=1)

    out2p = _s3(src, dst, z2e, as2, ad2, m2vec)[:, :_N, :]

    bias2b = jnp.broadcast_to(bias2[None, :], (8, 128))
    return _k4(out2p, z2e, aux2, m2arr, bias2b)


# speedup vs baseline: 17.7117x; 1.6211x over previous
import functools

import jax
import jax.numpy as jnp
from jax import lax
from jax.experimental import pallas as pl
from jax.experimental.pallas import tpu as pltpu
from jax.experimental.pallas import tpu_sc as plsc

_N = 10000
_E = 320000
_D = 128
_H = 8
_HID = 128

_NTILES = 32
_EPT = _E // _NTILES
_B = 80
_NCHUNK = _EPT // _B
_NPAD = 10240
_RPT = _NPAD // 16
_W2COLS = 144

_SC_PARAMS = pltpu.CompilerParams(
    use_tc_tiling_on_sc=False, needs_layout_passes=False)


def _mesh():
    return plsc.VectorSubcoreMesh(core_axis_name="c", subcore_axis_name="s")


def _mm2_body(x_ref, w1_ref, w2_ref, o1_ref, o2_ref):
    x = x_ref[...]
    o1_ref[...] = jnp.dot(x, w1_ref[...], preferred_element_type=jnp.float32)
    o2_ref[...] = jnp.dot(x, w2_ref[...], preferred_element_type=jnp.float32)


def _matmul2(x, w1, w2, blk_m=2000):
    m, k = x.shape
    return pl.pallas_call(
        _mm2_body,
        grid=(m // blk_m,),
        in_specs=[
            pl.BlockSpec((blk_m, k), lambda i: (i, 0)),
            pl.BlockSpec((k, w1.shape[1]), lambda i: (0, 0)),
            pl.BlockSpec((k, w2.shape[1]), lambda i: (0, 0)),
        ],
        out_specs=[
            pl.BlockSpec((blk_m, w1.shape[1]), lambda i: (i, 0)),
            pl.BlockSpec((blk_m, w2.shape[1]), lambda i: (i, 0)),
        ],
        out_shape=[
            jax.ShapeDtypeStruct((m, w1.shape[1]), jnp.float32),
            jax.ShapeDtypeStruct((m, w2.shape[1]), jnp.float32),
        ],
    )(x, w1, w2)


def _s1_body(src_h, dst_h, asd_h, m_h, zr_h,
             ext_h, s1p_h,
             srcb, dstb, rs, rd, exs, exT, mv, acc, sem):
    cid = lax.axis_index("c")
    sid = lax.axis_index("s")
    wid = sid * 2 + cid
    perm = (lax.iota(jnp.int32, 16) % 8) + 8

    pltpu.sync_copy(m_h, mv)
    pltpu.sync_copy(zr_h, exs)
    for i in range(_RPT // _B):
        pltpu.sync_copy(exs, acc.at[pl.ds(sid * _RPT + i * _B, _B)])
    plsc.subcore_barrier()

    def chunk(c, carry):
        base = wid * _EPT + c * _B
        pltpu.sync_copy(src_h.at[pl.ds(base, _B)], srcb)
        pltpu.sync_copy(dst_h.at[pl.ds(base, _B)], dstb)
        pltpu.async_copy(asd_h.at[srcb], rs, sem).wait()
        pltpu.async_copy(asd_h.at[dstb], rd, sem).wait()
        mvv = mv[...]

        def jbody(j, carry2):
            a = rs[j, :]
            bp = plsc.load_gather(rd, [jnp.zeros((16,), jnp.int32) + j, perm])
            e = a + bp
            e = jnp.where(e >= 0, e, 0.2 * e)
            ex = jnp.exp(e - mvv)
            exs[j, :] = ex
            plsc.store_scatter(
                exT, [lax.iota(jnp.int32, 16),
                      jnp.zeros((16,), jnp.int32) + (c * _B + j)],
                ex, mask=lax.iota(jnp.int32, 16) < 8)
            return carry2

        lax.fori_loop(0, _B, jbody, 0)
        pltpu.sync_copy(exs, acc.at[dstb], add=True)
        return carry

    lax.fori_loop(0, _NCHUNK, chunk, 0)
    plsc.subcore_barrier()

    pltpu.sync_copy(exT, ext_h.at[wid])
    for i in range(_RPT // _B):
        sl = pl.ds(sid * _RPT + i * _B, _B)
        pltpu.sync_copy(acc.at[sl], rs)
        pltpu.sync_copy(rs, s1p_h.at[cid, sl])


def _s1(src, dst, asd, m1vec):
    zr = jnp.zeros((_B, 16), jnp.float32)
    f = pl.kernel(
        _s1_body,
        out_type=[
            jax.ShapeDtypeStruct((_NTILES, _H, _EPT), jnp.float32),
            jax.ShapeDtypeStruct((2, _NPAD, 16), jnp.float32),
        ],
        mesh=_mesh(),
        scratch_types=[
            pltpu.VMEM((_B,), jnp.int32),
            pltpu.VMEM((_B,), jnp.int32),
            pltpu.VMEM((_B, 16), jnp.float32),
            pltpu.VMEM((_B, 16), jnp.float32),
            pltpu.VMEM((_B, 16), jnp.float32),
            pltpu.VMEM((_H, _EPT), jnp.float32),
            pltpu.VMEM((16,), jnp.float32),
            pltpu.VMEM_SHARED((_NPAD, 16), jnp.float32),
            pltpu.SemaphoreType.DMA,
        ],
        compiler_params=_SC_PARAMS,
    )
    return f(src, dst, asd, m1vec, zr)


def _s2_body(src3_h, dst3_h, h1r_h, ext4_h, zr_h,
             out_h,
             srcst, dstst, gidx0, gidx1, exb0, exb1, rows0, rows1,
             acc, sem0, sem1):
    cid = lax.axis_index("c")
    sid = lax.axis_index("s")
    wid = sid * 2 + cid

    pltpu.sync_copy(src3_h.at[wid], srcst)
    pltpu.sync_copy(dst3_h.at[wid], dstst)

    for h in range(_H):
        pltpu.sync_copy(zr_h, rows0)
        for i in range(_RPT // _B):
            pltpu.sync_copy(rows0, acc.at[pl.ds(sid * _RPT + i * _B, _B)])
        plsc.subcore_barrier()

        def issue(c, exb, gidx, rows, sem):
            pltpu.sync_copy(ext4_h.at[wid, h, c], exb)
            for k in range(_B // 16):
                sl = pl.ds(k * 16, 16)
                gidx[sl] = srcst[c, sl] * 8 + h
            pltpu.async_copy(h1r_h.at[gidx], rows, sem)

        def process(c, exb, gidx, rows, sem):
            pltpu.make_async_copy(h1r_h.at[gidx], rows, sem).wait()

            def jbody(j, carry2):
                w = plsc.load_gather(exb, [jnp.zeros((16,), jnp.int32) + j])
                for k in range(_D // 16):
                    sl = pl.ds(k * 16, 16)
                    rows[j, sl] = rows[j, sl] * w
                return carry2

            lax.fori_loop(0, _B, jbody, 0)
            pltpu.sync_copy(rows, acc.at[dstst.at[c]], add=True)

        issue(0, exb0, gidx0, rows0, sem0)

        def pair(i, carry):
            c0 = 2 * i
            issue(c0 + 1, exb1, gidx1, rows1, sem1)
            process(c0, exb0, gidx0, rows0, sem0)
            issue(c0 + 2, exb0, gidx0, rows0, sem0)
            process(c0 + 1, exb1, gidx1, rows1, sem1)
            return carry

        lax.fori_loop(0, (_NCHUNK - 1) // 2, pair, 0)
        process(_NCHUNK - 1, exb0, gidx0, rows0, sem0)
        plsc.subcore_barrier()

        for i in range(_RPT // _B):
            sl = pl.ds(sid * _RPT + i * _B, _B)
            pltpu.sync_copy(acc.at[sl], rows0)
            pltpu.sync_copy(rows0, out_h.at[cid, h, sl])
        plsc.subcore_barrier()


def _s2(src, dst, h1r, ext):
    zr = jnp.zeros((_B, _D), jnp.float32)
    src3 = src.reshape(_NTILES, _NCHUNK, _B)
    dst3 = dst.reshape(_NTILES, _NCHUNK, _B)
    ext4 = ext.reshape(_NTILES, _H, _NCHUNK, _B)
    f = pl.kernel(
        _s2_body,
        out_type=jax.ShapeDtypeStruct((2, _H, _NPAD, _D), jnp.float32),
        mesh=_mesh(),
        scratch_types=[
            pltpu.VMEM((_NCHUNK, _B), jnp.int32),
            pltpu.VMEM((_NCHUNK, _B), jnp.int32),
            pltpu.VMEM((_B,), jnp.int32),
            pltpu.VMEM((_B,), jnp.int32),
            pltpu.VMEM((_B,), jnp.float32),
            pltpu.VMEM((_B,), jnp.float32),
            pltpu.VMEM((_B, _D), jnp.float32),
            pltpu.VMEM((_B, _D), jnp.float32),
            pltpu.VMEM_SHARED((_NPAD, _D), jnp.float32),
            pltpu.SemaphoreType.DMA,
            pltpu.SemaphoreType.DMA,
        ],
        compiler_params=_SC_PARAMS,
    )
    return f(src3, dst3, h1r, ext4, zr)


def _k3a_body(p_ref, h1_ref, aux_ref, s_ref, m1_ref, b1_ref,
              o_ref, sums_ref, sq_ref):
    aux = aux_ref[...]
    a_s = aux[:, :_H]
    a_d = aux[:, _H:2 * _H]
    el = a_s + a_d
    el = jnp.where(el >= 0, el, 0.2 * el)
    exw = jnp.exp(el - m1_ref[...][0:1, :])
    s_tot = s_ref[0][:, :_H] + s_ref[1][:, :_H] + exw
    inv = 1.0 / s_tot
    parts = []
    for h in range(_H):
        num = (p_ref[0, h] + p_ref[1, h]
               + exw[:, h:h + 1] * h1_ref[:, h * _HID:(h + 1) * _HID])
        parts.append(num * inv[:, h:h + 1])
    hout = jnp.concatenate(parts, axis=1) + b1_ref[...][0:1, :]
    o_ref[...] = hout
    cs = jnp.broadcast_to(jnp.sum(hout, axis=0)[None, :], (8, _H * _HID))
    css = jnp.broadcast_to(jnp.sum(hout * hout, axis=0)[None, :],
                           (8, _H * _HID))

    @pl.when(pl.program_id(0) == 0)
    def _():
        sums_ref[...] = cs
        sq_ref[...] = css

    @pl.when(pl.program_id(0) > 0)
    def _():
        sums_ref[...] += cs
        sq_ref[...] += css


def _k3a(out1p, h1, aux, s1p, m1b, bias1b, blk=1000):
    return pl.pallas_call(
        _k3a_body,
        grid=(_N // blk,),
        in_specs=[
            pl.BlockSpec((2, _H, blk, _D), lambda i: (0, 0, i, 0)),
            pl.BlockSpec((blk, _H * _HID), lambda i: (i, 0)),
            pl.BlockSpec((blk, _D), lambda i: (i, 0)),
            pl.BlockSpec((2, blk, 16), lambda i: (0, i, 0)),
            pl.BlockSpec((8, _H), lambda i: (0, 0)),
            pl.BlockSpec((8, _H * _HID), lambda i: (0, 0)),
        ],
        out_specs=[
            pl.BlockSpec((blk, _H * _HID), lambda i: (i, 0)),
            pl.BlockSpec((8, _H * _HID), lambda i: (0, 0)),
            pl.BlockSpec((8, _H * _HID), lambda i: (0, 0)),
        ],
        out_shape=[
            jax.ShapeDtypeStruct((_N, _H * _HID), jnp.float32),
            jax.ShapeDtypeStruct((8, _H * _HID), jnp.float32),
            jax.ShapeDtypeStruct((8, _H * _HID), jnp.float32),
        ],
    )(out1p, h1, aux, s1p, m1b, bias1b)


def _k3b_body(h_ref, sc_ref, sh_ref, w_ref, o_ref):
    hb = h_ref[...] * sc_ref[...][0:1, :] + sh_ref[...][0:1, :]
    he = jnp.where(hb > 0, hb, jnp.exp(hb) - 1.0)
    o_ref[...] = jnp.dot(he, w_ref[...], preferred_element_type=jnp.float32)


def _k3b(h, scale8, shift8, w2e, blk=1000):
    k = h.shape[1]
    n = w2e.shape[1]
    return pl.pallas_call(
        _k3b_body,
        grid=(_N // blk,),
        in_specs=[
            pl.BlockSpec((blk, k), lambda i: (i, 0)),
            pl.BlockSpec((8, k), lambda i: (0, 0)),
            pl.BlockSpec((8, k), lambda i: (0, 0)),
            pl.BlockSpec((k, n), lambda i: (0, 0)),
        ],
        out_specs=pl.BlockSpec((blk, n), lambda i: (i, 0)),
        out_shape=jax.ShapeDtypeStruct((_N, n), jnp.float32),
    )(h, scale8, shift8, w2e)


def _s3_body(src_h, dst_h, z2e_h, as2_h, ad2_h, m2_h, zr_h,
             out_h,
             as2v, ad2v, srcb, dstb, exb, rows, mv, acc, sem):
    cid = lax.axis_index("c")
    sid = lax.axis_index("s")
    wid = sid * 2 + cid

    pltpu.sync_copy(as2_h, as2v)
    pltpu.sync_copy(ad2_h, ad2v)
    pltpu.sync_copy(m2_h, mv)
    pltpu.sync_copy(zr_h, rows)
    for i in range(_RPT // _B):
        pltpu.sync_copy(rows, acc.at[pl.ds(sid * _RPT + i * _B, _B)])
    plsc.subcore_barrier()

    def chunk(c, carry):
        base = wid * _EPT + c * _B
        pltpu.sync_copy(src_h.at[pl.ds(base, _B)], srcb)
        pltpu.sync_copy(dst_h.at[pl.ds(base, _B)], dstb)
        mvv = mv[...]
        for k in range(_B // 16):
            sl = pl.ds(k * 16, 16)
            vs = plsc.load_gather(as2v, [srcb[sl]])
            vd = plsc.load_gather(ad2v, [dstb[sl]])
            e = vs + vd
            e = jnp.where(e >= 0, e, 0.2 * e)
            exb[sl] = jnp.exp(e - mvv)
        pltpu.async_copy(z2e_h.at[srcb], rows, sem).wait()

        def jbody(j, carry2):
            w = plsc.load_gather(exb, [jnp.zeros((16,), jnp.int32) + j])
            for k in range(_W2COLS // 16):
                sl = pl.ds(k * 16, 16)
                rows[j, sl] = rows[j, sl] * w
            return carry2

        lax.fori_loop(0, _B, jbody, 0)
        pltpu.sync_copy(rows, acc.at[dstb], add=True)
        return carry

    lax.fori_loop(0, _NCHUNK, chunk, 0)
    plsc.subcore_barrier()

    for i in range(_RPT // _B):
        sl = pl.ds(sid * _RPT + i * _B, _B)
        pltpu.sync_copy(acc.at[sl], rows)
        pltpu.sync_copy(rows, out_h.at[cid, sl])


def _s3(src, dst, z2e, as2, ad2, m2vec):
    zr = jnp.zeros((_B, _W2COLS), jnp.float32)
    f = pl.kernel(
        _s3_body,
        out_type=jax.ShapeDtypeStruct((2, _NPAD, _W2COLS), jnp.float32),
        mesh=_mesh(),
        scratch_types=[
            pltpu.VMEM((_N,), jnp.float32),
            pltpu.VMEM((_N,), jnp.float32),
            pltpu.VMEM((_B,), jnp.int32),
            pltpu.VMEM((_B,), jnp.int32),
            pltpu.VMEM((_B,), jnp.float32),
            pltpu.VMEM((_B, _W2COLS), jnp.float32),
            pltpu.VMEM((16,), jnp.float32),
            pltpu.VMEM_SHARED((_NPAD, _W2COLS), jnp.float32),
            pltpu.SemaphoreType.DMA,
        ],
        compiler_params=_SC_PARAMS,
    )
    return f(src, dst, z2e, as2, ad2, m2vec, zr)


def _k4_body(p_ref, z_ref, aux_ref, m2_ref, b2_ref, o_ref):
    p0 = p_ref[0]
    p1 = p_ref[1]
    z = z_ref[...]
    a = aux_ref[...]
    m2 = m2_ref[...][0:1, 0:1]
    el = a[:, 0:1] + a[:, 1:2]
    el = jnp.where(el >= 0, el, 0.2 * el)
    exw = jnp.exp(el - m2)
    num = p0[:, :_D] + p1[:, :_D] + exw * z[:, :_D]
    den = p0[:, _D:_D + 1] + p1[:, _D:_D + 1] + exw
    o_ref[...] = num / den + b2_ref[...][0:1, :]


def _k4(out2p, z2e, aux2, m2arr, bias2b, blk=1000):
    return pl.pallas_call(
        _k4_body,
        grid=(_N // blk,),
        in_specs=[
            pl.BlockSpec((2, blk, _W2COLS), lambda i: (0, i, 0)),
            pl.BlockSpec((blk, _W2COLS), lambda i: (i, 0)),
            pl.BlockSpec((blk, 16), lambda i: (i, 0)),
            pl.BlockSpec((8, 128), lambda i: (0, 0)),
            pl.BlockSpec((8, 128), lambda i: (0, 0)),
        ],
        out_specs=pl.BlockSpec((blk, _D), lambda i: (i, 0)),
        out_shape=jax.ShapeDtypeStruct((_N, _D), jnp.float32),
    )(out2p, z2e, aux2, m2arr, bias2b)


def kernel(x, edge_index, W1, att_src1, att_dst1, bias1, gamma, beta, W2,
           att_src2, att_dst2, bias2):
    src, dst = edge_index[0], edge_index[1]

    W1r = W1.reshape(_D, _H, _HID)
    As1 = jnp.einsum('dhc,hc->dh', W1r, att_src1)
    Ad1 = jnp.einsum('dhc,hc->dh', W1r, att_dst1)
    pad1 = jnp.zeros((_D, 128 - 2 * _H), jnp.float32)
    Waux1 = jnp.concatenate([As1, Ad1, pad1], axis=1)

    h1, aux = _matmul2(x, W1, Waux1)
    asd = aux[:, :16]
    m1 = jnp.max(aux[:, :_H], axis=0) + jnp.max(aux[:, _H:2 * _H], axis=0)
    m1vec = jnp.concatenate([m1, jnp.full((8,), 1e30, jnp.float32)])
    m1b = jnp.broadcast_to(m1[None, :], (8, _H))

    ext, s1p = _s1(src, dst, asd, m1vec)
    h1r = h1.reshape(_N * _H, _HID)
    out1p = _s2(src, dst, h1r, ext)

    bias1b = jnp.broadcast_to(bias1[None, :], (8, _H * _HID))
    h, sums, sq = _k3a(out1p[:, :, :_N, :], h1, aux, s1p[:, :_N, :],
                       m1b, bias1b)

    mean = sums[0] / _N
    var = sq[0] / _N - mean * mean
    scale = gamma / jnp.sqrt(var + 1e-5)
    shift = beta - mean * scale
    scale8 = jnp.broadcast_to(scale[None, :], (8, _H * _HID))
    shift8 = jnp.broadcast_to(shift[None, :], (8, _H * _HID))

    W2r = W2.reshape(_H * _HID, 1, _D)
    As2 = jnp.einsum('dhc,hc->dh', W2r, att_src2)
    Ad2 = jnp.einsum('dhc,hc->dh', W2r, att_dst2)
    pad2 = jnp.zeros((_H * _HID, 14), jnp.float32)
    W2e = jnp.concatenate([W2, As2, Ad2, pad2], axis=1)

    h2e = _k3b(h, scale8, shift8, W2e)
    z2 = h2e[:, :_D]
    as2 = h2e[:, _D]
    ad2 = h2e[:, _D + 1]
    aux2 = h2e[:, _D:_D + 16]

    m2 = jnp.max(as2) + jnp.max(ad2)
    m2vec = jnp.full((16,), m2, jnp.float32)
    m2arr = jnp.full((8, 128), m2, jnp.float32)

    onescol = (lax.broadcasted_iota(jnp.int32, (_N, _W2COLS - _D), 1)
               == 0).astype(jnp.float32)
    z2e = jnp.concatenate([z2, onescol], axis=1)

    out2p = _s3(src, dst, z2e, as2, ad2, m2vec)[:, :_N, :]

    bias2b = jnp.broadcast_to(bias2[None, :], (8, 128))
    return _k4(out2p, z2e, aux2, m2arr, bias2b)


# speedup vs baseline: 19.7164x; 1.1132x over previous
import functools

import jax
import jax.numpy as jnp
from jax import lax
from jax.experimental import pallas as pl
from jax.experimental.pallas import tpu as pltpu
from jax.experimental.pallas import tpu_sc as plsc

_N = 10000
_E = 320000
_D = 128
_H = 8
_HID = 128

_NTILES = 32
_EPT = _E // _NTILES
_B = 80
_NCHUNK = _EPT // _B
_NPAD = 10240
_RPT = _NPAD // 16
_W2COLS = 144

_SC_PARAMS = pltpu.CompilerParams(
    use_tc_tiling_on_sc=False, needs_layout_passes=False)


def _mesh():
    return plsc.VectorSubcoreMesh(core_axis_name="c", subcore_axis_name="s")


def _mm2_body(x_ref, w1_ref, w2_ref, o1_ref, o2_ref):
    x = x_ref[...]
    o1_ref[...] = jnp.dot(x, w1_ref[...], preferred_element_type=jnp.float32)
    o2_ref[...] = jnp.dot(x, w2_ref[...], preferred_element_type=jnp.float32)


def _matmul2(x, w1, w2, blk_m=2000):
    m, k = x.shape
    return pl.pallas_call(
        _mm2_body,
        grid=(m // blk_m,),
        in_specs=[
            pl.BlockSpec((blk_m, k), lambda i: (i, 0)),
            pl.BlockSpec((k, w1.shape[1]), lambda i: (0, 0)),
            pl.BlockSpec((k, w2.shape[1]), lambda i: (0, 0)),
        ],
        out_specs=[
            pl.BlockSpec((blk_m, w1.shape[1]), lambda i: (i, 0)),
            pl.BlockSpec((blk_m, w2.shape[1]), lambda i: (i, 0)),
        ],
        out_shape=[
            jax.ShapeDtypeStruct((m, w1.shape[1]), jnp.float32),
            jax.ShapeDtypeStruct((m, w2.shape[1]), jnp.float32),
        ],
    )(x, w1, w2)


def _s1_body(src3_h, dst3_h, asd_h, m_h, zr_h,
             ext_h, s1p_h,
             srcst, dstst, rs0, rd0, rs1, rd1, exs, exT, mv, acc,
             sem0, sem1):
    cid = lax.axis_index("c")
    sid = lax.axis_index("s")
    wid = sid * 2 + cid
    perm = (lax.iota(jnp.int32, 16) % 8) + 8

    pltpu.sync_copy(src3_h.at[wid], srcst)
    pltpu.sync_copy(dst3_h.at[wid], dstst)
    pltpu.sync_copy(m_h, mv)
    pltpu.sync_copy(zr_h, exs)
    for i in range(_RPT // _B):
        pltpu.sync_copy(exs, acc.at[pl.ds(sid * _RPT + i * _B, _B)])
    plsc.subcore_barrier()

    def issue(c, rs_, rd_, sem_):
        pltpu.async_copy(asd_h.at[srcst.at[c]], rs_, sem_)
        pltpu.async_copy(asd_h.at[dstst.at[c]], rd_, sem_)

    def process(c, rs_, rd_, sem_):
        pltpu.make_async_copy(asd_h.at[srcst.at[c]], rs_, sem_).wait()
        pltpu.make_async_copy(asd_h.at[dstst.at[c]], rd_, sem_).wait()
        mvv = mv[...]

        def jbody(j, carry2):
            a = rs_[j, :]
            bp = plsc.load_gather(rd_, [jnp.zeros((16,), jnp.int32) + j, perm])
            e = a + bp
            e = jnp.where(e >= 0, e, 0.2 * e)
            ex = jnp.exp(e - mvv)
            exs[j, :] = ex
            plsc.store_scatter(
                exT, [lax.iota(jnp.int32, 16),
                      jnp.zeros((16,), jnp.int32) + (c * _B + j)],
                ex, mask=lax.iota(jnp.int32, 16) < 8)
            return carry2

        lax.fori_loop(0, _B, jbody, 0)
        pltpu.sync_copy(exs, acc.at[dstst.at[c]], add=True)

    issue(0, rs0, rd0, sem0)

    def pair(i, carry):
        c0 = 2 * i
        issue(c0 + 1, rs1, rd1, sem1)
        process(c0, rs0, rd0, sem0)
        issue(c0 + 2, rs0, rd0, sem0)
        process(c0 + 1, rs1, rd1, sem1)
        return carry

    lax.fori_loop(0, (_NCHUNK - 1) // 2, pair, 0)
    process(_NCHUNK - 1, rs0, rd0, sem0)
    plsc.subcore_barrier()

    pltpu.sync_copy(exT, ext_h.at[wid])
    for i in range(_RPT // _B):
        sl = pl.ds(sid * _RPT + i * _B, _B)
        pltpu.sync_copy(acc.at[sl], rs0)
        pltpu.sync_copy(rs0, s1p_h.at[cid, sl])


def _s1(src, dst, asd, m1vec):
    zr = jnp.zeros((_B, 16), jnp.float32)
    src3 = src.reshape(_NTILES, _NCHUNK, _B)
    dst3 = dst.reshape(_NTILES, _NCHUNK, _B)
    f = pl.kernel(
        _s1_body,
        out_type=[
            jax.ShapeDtypeStruct((_NTILES, _H, _EPT), jnp.float32),
            jax.ShapeDtypeStruct((2, _NPAD, 16), jnp.float32),
        ],
        mesh=_mesh(),
        scratch_types=[
            pltpu.VMEM((_NCHUNK, _B), jnp.int32),
            pltpu.VMEM((_NCHUNK, _B), jnp.int32),
            pltpu.VMEM((_B, 16), jnp.float32),
            pltpu.VMEM((_B, 16), jnp.float32),
            pltpu.VMEM((_B, 16), jnp.float32),
            pltpu.VMEM((_B, 16), jnp.float32),
            pltpu.VMEM((_B, 16), jnp.float32),
            pltpu.VMEM((_H, _EPT), jnp.float32),
            pltpu.VMEM((16,), jnp.float32),
            pltpu.VMEM_SHARED((_NPAD, 16), jnp.float32),
            pltpu.SemaphoreType.DMA,
            pltpu.SemaphoreType.DMA,
        ],
        compiler_params=_SC_PARAMS,
    )
    return f(src3, dst3, asd, m1vec, zr)


def _s2_body(src3_h, dst3_h, h1r_h, ext4_h, zr_h,
             out_h,
             srcst, dstst, gidx0, gidx1, exb0, exb1, rows0, rows1,
             acc, sem0, sem1):
    cid = lax.axis_index("c")
    sid = lax.axis_index("s")
    wid = sid * 2 + cid

    pltpu.sync_copy(src3_h.at[wid], srcst)
    pltpu.sync_copy(dst3_h.at[wid], dstst)

    for h in range(_H):
        pltpu.sync_copy(zr_h, rows0)
        for i in range(_RPT // _B):
            pltpu.sync_copy(rows0, acc.at[pl.ds(sid * _RPT + i * _B, _B)])
        plsc.subcore_barrier()

        def issue(c, exb, gidx, rows, sem):
            pltpu.sync_copy(ext4_h.at[wid, h, c], exb)
            for k in range(_B // 16):
                sl = pl.ds(k * 16, 16)
                gidx[sl] = srcst[c, sl] * 8 + h
            pltpu.async_copy(h1r_h.at[gidx], rows, sem)

        def process(c, exb, gidx, rows, sem):
            pltpu.make_async_copy(h1r_h.at[gidx], rows, sem).wait()

            def jbody(j, carry2):
                w = plsc.load_gather(exb, [jnp.zeros((16,), jnp.int32) + j])
                for k in range(_D // 16):
                    sl = pl.ds(k * 16, 16)
                    rows[j, sl] = rows[j, sl] * w
                return carry2

            lax.fori_loop(0, _B, jbody, 0)
            pltpu.sync_copy(rows, acc.at[dstst.at[c]], add=True)

        issue(0, exb0, gidx0, rows0, sem0)

        def pair(i, carry):
            c0 = 2 * i
            issue(c0 + 1, exb1, gidx1, rows1, sem1)
            process(c0, exb0, gidx0, rows0, sem0)
            issue(c0 + 2, exb0, gidx0, rows0, sem0)
            process(c0 + 1, exb1, gidx1, rows1, sem1)
            return carry

        lax.fori_loop(0, (_NCHUNK - 1) // 2, pair, 0)
        process(_NCHUNK - 1, exb0, gidx0, rows0, sem0)
        plsc.subcore_barrier()

        for i in range(_RPT // _B):
            sl = pl.ds(sid * _RPT + i * _B, _B)
            pltpu.sync_copy(acc.at[sl], rows0)
            pltpu.sync_copy(rows0, out_h.at[cid, h, sl])
        plsc.subcore_barrier()


def _s2(src, dst, h1r, ext):
    zr = jnp.zeros((_B, _D), jnp.float32)
    src3 = src.reshape(_NTILES, _NCHUNK, _B)
    dst3 = dst.reshape(_NTILES, _NCHUNK, _B)
    ext4 = ext.reshape(_NTILES, _H, _NCHUNK, _B)
    f = pl.kernel(
        _s2_body,
        out_type=jax.ShapeDtypeStruct((2, _H, _NPAD, _D), jnp.float32),
        mesh=_mesh(),
        scratch_types=[
            pltpu.VMEM((_NCHUNK, _B), jnp.int32),
            pltpu.VMEM((_NCHUNK, _B), jnp.int32),
            pltpu.VMEM((_B,), jnp.int32),
            pltpu.VMEM((_B,), jnp.int32),
            pltpu.VMEM((_B,), jnp.float32),
            pltpu.VMEM((_B,), jnp.float32),
            pltpu.VMEM((_B, _D), jnp.float32),
            pltpu.VMEM((_B, _D), jnp.float32),
            pltpu.VMEM_SHARED((_NPAD, _D), jnp.float32),
            pltpu.SemaphoreType.DMA,
            pltpu.SemaphoreType.DMA,
        ],
        compiler_params=_SC_PARAMS,
    )
    return f(src3, dst3, h1r, ext4, zr)


def _k3a_body(p_ref, h1_ref, aux_ref, s_ref, m1_ref, b1_ref,
              o_ref, sums_ref, sq_ref):
    aux = aux_ref[...]
    a_s = aux[:, :_H]
    a_d = aux[:, _H:2 * _H]
    el = a_s + a_d
    el = jnp.where(el >= 0, el, 0.2 * el)
    exw = jnp.exp(el - m1_ref[...][0:1, :])
    s_tot = s_ref[0][:, :_H] + s_ref[1][:, :_H] + exw
    inv = 1.0 / s_tot
    parts = []
    for h in range(_H):
        num = (p_ref[0, h] + p_ref[1, h]
               + exw[:, h:h + 1] * h1_ref[:, h * _HID:(h + 1) * _HID])
        parts.append(num * inv[:, h:h + 1])
    hout = jnp.concatenate(parts, axis=1) + b1_ref[...][0:1, :]
    o_ref[...] = hout
    cs = jnp.broadcast_to(jnp.sum(hout, axis=0)[None, :], (8, _H * _HID))
    css = jnp.broadcast_to(jnp.sum(hout * hout, axis=0)[None, :],
                           (8, _H * _HID))

    @pl.when(pl.program_id(0) == 0)
    def _():
        sums_ref[...] = cs
        sq_ref[...] = css

    @pl.when(pl.program_id(0) > 0)
    def _():
        sums_ref[...] += cs
        sq_ref[...] += css


def _k3a(out1p, h1, aux, s1p, m1b, bias1b, blk=1000):
    return pl.pallas_call(
        _k3a_body,
        grid=(_N // blk,),
        in_specs=[
            pl.BlockSpec((2, _H, blk, _D), lambda i: (0, 0, i, 0)),
            pl.BlockSpec((blk, _H * _HID), lambda i: (i, 0)),
            pl.BlockSpec((blk, _D), lambda i: (i, 0)),
            pl.BlockSpec((2, blk, 16), lambda i: (0, i, 0)),
            pl.BlockSpec((8, _H), lambda i: (0, 0)),
            pl.BlockSpec((8, _H * _HID), lambda i: (0, 0)),
        ],
        out_specs=[
            pl.BlockSpec((blk, _H * _HID), lambda i: (i, 0)),
            pl.BlockSpec((8, _H * _HID), lambda i: (0, 0)),
            pl.BlockSpec((8, _H * _HID), lambda i: (0, 0)),
        ],
        out_shape=[
            jax.ShapeDtypeStruct((_N, _H * _HID), jnp.float32),
            jax.ShapeDtypeStruct((8, _H * _HID), jnp.float32),
            jax.ShapeDtypeStruct((8, _H * _HID), jnp.float32),
        ],
    )(out1p, h1, aux, s1p, m1b, bias1b)


def _k3b_body(h_ref, sc_ref, sh_ref, w_ref, o_ref):
    hb = h_ref[...] * sc_ref[...][0:1, :] + sh_ref[...][0:1, :]
    he = jnp.where(hb > 0, hb, jnp.exp(hb) - 1.0)
    o_ref[...] = jnp.dot(he, w_ref[...], preferred_element_type=jnp.float32)


def _k3b(h, scale8, shift8, w2e, blk=1000):
    k = h.shape[1]
    n = w2e.shape[1]
    return pl.pallas_call(
        _k3b_body,
        grid=(_N // blk,),
        in_specs=[
            pl.BlockSpec((blk, k), lambda i: (i, 0)),
            pl.BlockSpec((8, k), lambda i: (0, 0)),
            pl.BlockSpec((8, k), lambda i: (0, 0)),
            pl.BlockSpec((k, n), lambda i: (0, 0)),
        ],
        out_specs=pl.BlockSpec((blk, n), lambda i: (i, 0)),
        out_shape=jax.ShapeDtypeStruct((_N, n), jnp.float32),
    )(h, scale8, shift8, w2e)


def _s3_body(src3_h, dst3_h, z2e_h, ad2_h, m2_h, zr_h,
             out_h,
             ad2v, srcb0, srcb1, dstb0, dstb1, adb, rows0, rows1, mv, acc,
             sem0, sem1):
    cid = lax.axis_index("c")
    sid = lax.axis_index("s")
    wid = sid * 2 + cid

    pltpu.sync_copy(ad2_h, ad2v)
    pltpu.sync_copy(m2_h, mv)
    pltpu.sync_copy(zr_h, rows0)
    for i in range(_RPT // _B):
        pltpu.sync_copy(rows0, acc.at[pl.ds(sid * _RPT + i * _B, _B)])
    plsc.subcore_barrier()

    def issue(c, srcb_, dstb_, rows_, sem_):
        pltpu.sync_copy(src3_h.at[wid, c], srcb_)
        pltpu.sync_copy(dst3_h.at[wid, c], dstb_.at[0])
        pltpu.async_copy(z2e_h.at[srcb_], rows_, sem_)

    def process(c, srcb_, dstb_, rows_, sem_):
        pltpu.make_async_copy(z2e_h.at[srcb_], rows_, sem_).wait()
        mvv = mv[...]
        for k in range(_B // 16):
            sl = pl.ds(k * 16, 16)
            adb[sl] = plsc.load_gather(ad2v, [dstb_[0, sl]])

        def jbody(j, carry2):
            jv = jnp.zeros((16,), jnp.int32) + j
            wa = plsc.load_gather(rows_, [jv, jnp.full((16,), 129, jnp.int32)])
            wd = plsc.load_gather(adb, [jv])
            e = wa + wd
            e = jnp.where(e >= 0, e, 0.2 * e)
            w = jnp.exp(e - mvv)
            for k in range(_W2COLS // 16):
                sl = pl.ds(k * 16, 16)
                rows_[j, sl] = rows_[j, sl] * w
            return carry2

        lax.fori_loop(0, _B, jbody, 0)
        pltpu.sync_copy(rows_, acc.at[dstb_.at[0]], add=True)

    issue(0, srcb0, dstb0, rows0, sem0)

    def pair(i, carry):
        c0 = 2 * i
        issue(c0 + 1, srcb1, dstb1, rows1, sem1)
        process(c0, srcb0, dstb0, rows0, sem0)
        issue(c0 + 2, srcb0, dstb0, rows0, sem0)
        process(c0 + 1, srcb1, dstb1, rows1, sem1)
        return carry

    lax.fori_loop(0, (_NCHUNK - 1) // 2, pair, 0)
    process(_NCHUNK - 1, srcb0, dstb0, rows0, sem0)
    plsc.subcore_barrier()

    for i in range(_RPT // _B):
        sl = pl.ds(sid * _RPT + i * _B, _B)
        pltpu.sync_copy(acc.at[sl], rows0)
        pltpu.sync_copy(rows0, out_h.at[cid, sl])


def _s3(src, dst, z2e, ad2, m2vec):
    zr = jnp.zeros((_B, _W2COLS), jnp.float32)
    src3 = src.reshape(_NTILES, _NCHUNK, _B)
    dst3 = dst.reshape(_NTILES, _NCHUNK, _B)
    f = pl.kernel(
        _s3_body,
        out_type=jax.ShapeDtypeStruct((2, _NPAD, _W2COLS), jnp.float32),
        mesh=_mesh(),
        scratch_types=[
            pltpu.VMEM((_N,), jnp.float32),
            pltpu.VMEM((_B,), jnp.int32),
            pltpu.VMEM((_B,), jnp.int32),
            pltpu.VMEM((1, _B), jnp.int32),
            pltpu.VMEM((1, _B), jnp.int32),
            pltpu.VMEM((_B,), jnp.float32),
            pltpu.VMEM((_B, _W2COLS), jnp.float32),
            pltpu.VMEM((_B, _W2COLS), jnp.float32),
            pltpu.VMEM((16,), jnp.float32),
            pltpu.VMEM_SHARED((_NPAD, _W2COLS), jnp.float32),
            pltpu.SemaphoreType.DMA,
            pltpu.SemaphoreType.DMA,
        ],
        compiler_params=_SC_PARAMS,
    )
    return f(src3, dst3, z2e, ad2, m2vec, zr)


def _k4_body(p_ref, z_ref, aux_ref, m2_ref, b2_ref, o_ref):
    p0 = p_ref[0]
    p1 = p_ref[1]
    z = z_ref[...]
    a = aux_ref[...]
    m2 = m2_ref[...][0:1, 0:1]
    el = a[:, 0:1] + a[:, 1:2]
    el = jnp.where(el >= 0, el, 0.2 * el)
    exw = jnp.exp(el - m2)
    num = p0[:, :_D] + p1[:, :_D] + exw * z[:, :_D]
    den = p0[:, _D:_D + 1] + p1[:, _D:_D + 1] + exw
    o_ref[...] = num / den + b2_ref[...][0:1, :]


def _k4(out2p, z2e, aux2, m2arr, bias2b, blk=1000):
    return pl.pallas_call(
        _k4_body,
        grid=(_N // blk,),
        in_specs=[
            pl.BlockSpec((2, blk, _W2COLS), lambda i: (0, i, 0)),
            pl.BlockSpec((blk, _W2COLS), lambda i: (i, 0)),
            pl.BlockSpec((blk, 16), lambda i: (i, 0)),
            pl.BlockSpec((8, 128), lambda i: (0, 0)),
            pl.BlockSpec((8, 128), lambda i: (0, 0)),
        ],
        out_specs=pl.BlockSpec((blk, _D), lambda i: (i, 0)),
        out_shape=jax.ShapeDtypeStruct((_N, _D), jnp.float32),
    )(out2p, z2e, aux2, m2arr, bias2b)


def kernel(x, edge_index, W1, att_src1, att_dst1, bias1, gamma, beta, W2,
           att_src2, att_dst2, bias2):
    src, dst = edge_index[0], edge_index[1]

    W1r = W1.reshape(_D, _H, _HID)
    As1 = jnp.einsum('dhc,hc->dh', W1r, att_src1)
    Ad1 = jnp.einsum('dhc,hc->dh', W1r, att_dst1)
    pad1 = jnp.zeros((_D, 128 - 2 * _H), jnp.float32)
    Waux1 = jnp.concatenate([As1, Ad1, pad1], axis=1)

    h1, aux = _matmul2(x, W1, Waux1)
    asd = aux[:, :16]
    m1 = jnp.max(aux[:, :_H], axis=0) + jnp.max(aux[:, _H:2 * _H], axis=0)
    m1vec = jnp.concatenate([m1, jnp.full((8,), 1e30, jnp.float32)])
    m1b = jnp.broadcast_to(m1[None, :], (8, _H))

    ext, s1p = _s1(src, dst, asd, m1vec)
    h1r = h1.reshape(_N * _H, _HID)
    out1p = _s2(src, dst, h1r, ext)

    bias1b = jnp.broadcast_to(bias1[None, :], (8, _H * _HID))
    h, sums, sq = _k3a(out1p[:, :, :_N, :], h1, aux, s1p[:, :_N, :],
                       m1b, bias1b)

    mean = sums[0] / _N
    var = sq[0] / _N - mean * mean
    scale = gamma / jnp.sqrt(var + 1e-5)
    shift = beta - mean * scale
    scale8 = jnp.broadcast_to(scale[None, :], (8, _H * _HID))
    shift8 = jnp.broadcast_to(shift[None, :], (8, _H * _HID))

    W2r = W2.reshape(_H * _HID, 1, _D)
    As2 = jnp.einsum('dhc,hc->dh', W2r, att_src2)
    Ad2 = jnp.einsum('dhc,hc->dh', W2r, att_dst2)
    pad2 = jnp.zeros((_H * _HID, 14), jnp.float32)
    W2e = jnp.concatenate([W2, As2, Ad2, pad2], axis=1)

    h2e = _k3b(h, scale8, shift8, W2e)
    z2 = h2e[:, :_D]
    as2 = h2e[:, _D]
    ad2 = h2e[:, _D + 1]
    aux2 = h2e[:, _D:_D + 16]

    m2 = jnp.max(as2) + jnp.max(ad2)
    m2vec = jnp.full((16,), m2, jnp.float32)
    m2arr = jnp.full((8, 128), m2, jnp.float32)

    extra = jnp.concatenate(
        [jnp.ones((_N, 1), jnp.float32), as2[:, None],
         jnp.zeros((_N, _W2COLS - _D - 2), jnp.float32)], axis=1)
    z2e = jnp.concatenate([z2, extra], axis=1)

    out2p = _s3(src, dst, z2e, ad2, m2vec)[:, :_N, :]

    bias2b = jnp.broadcast_to(bias2[None, :], (8, 128))
    return _k4(out2p, z2e, aux2, m2arr, bias2b)


# speedup vs baseline: 25.8116x; 1.3091x over previous
import functools

import jax
import jax.numpy as jnp
from jax import lax
from jax.experimental import pallas as pl
from jax.experimental.pallas import tpu as pltpu
from jax.experimental.pallas import tpu_sc as plsc

_N = 10000
_E = 320000
_D = 128
_H = 8
_HID = 128

_NTILES = 32
_EPT = _E // _NTILES
_B = 80
_NCHUNK = _EPT // _B
_NPAD = 10240
_RPT = _NPAD // 16
_W2COLS = 144

_SC_PARAMS = pltpu.CompilerParams(
    use_tc_tiling_on_sc=False, needs_layout_passes=False)


def _mesh():
    return plsc.VectorSubcoreMesh(core_axis_name="c", subcore_axis_name="s")


def _mm2_body(x_ref, w1_ref, w2_ref, o1_ref, o2_ref):
    x = x_ref[...]
    o1_ref[...] = jnp.dot(x, w1_ref[...], preferred_element_type=jnp.float32)
    o2_ref[...] = jnp.dot(x, w2_ref[...], preferred_element_type=jnp.float32)


def _matmul2(x, w1, w2, blk_m=2000):
    m, k = x.shape
    return pl.pallas_call(
        _mm2_body,
        grid=(m // blk_m,),
        in_specs=[
            pl.BlockSpec((blk_m, k), lambda i: (i, 0)),
            pl.BlockSpec((k, w1.shape[1]), lambda i: (0, 0)),
            pl.BlockSpec((k, w2.shape[1]), lambda i: (0, 0)),
        ],
        out_specs=[
            pl.BlockSpec((blk_m, w1.shape[1]), lambda i: (i, 0)),
            pl.BlockSpec((blk_m, w2.shape[1]), lambda i: (i, 0)),
        ],
        out_shape=[
            jax.ShapeDtypeStruct((m, w1.shape[1]), jnp.float32),
            jax.ShapeDtypeStruct((m, w2.shape[1]), jnp.float32),
        ],
    )(x, w1, w2)


def _s1_body(src3_h, dst3_h, asd_h, m_h, zr_h,
             ext_h, s1p_h,
             srcst, dstst, rs0, rd0, rs1, rd1, exs, exT, mv, acc,
             sem0, sem1):
    cid = lax.axis_index("c")
    sid = lax.axis_index("s")
    wid = sid * 2 + cid
    perm = (lax.iota(jnp.int32, 16) % 8) + 8

    pltpu.sync_copy(src3_h.at[wid], srcst)
    pltpu.sync_copy(dst3_h.at[wid], dstst)
    pltpu.sync_copy(m_h, mv)
    pltpu.sync_copy(zr_h, exs)
    for i in range(_RPT // _B):
        pltpu.sync_copy(exs, acc.at[pl.ds(sid * _RPT + i * _B, _B)])
    plsc.subcore_barrier()

    def issue(c, rs_, rd_, sem_):
        pltpu.async_copy(asd_h.at[srcst.at[c]], rs_, sem_)
        pltpu.async_copy(asd_h.at[dstst.at[c]], rd_, sem_)

    def process(c, rs_, rd_, sem_):
        pltpu.make_async_copy(asd_h.at[srcst.at[c]], rs_, sem_).wait()
        pltpu.make_async_copy(asd_h.at[dstst.at[c]], rd_, sem_).wait()
        mvv = mv[...]

        @plsc.parallel_loop(0, _B, unroll=4)
        def jbody(j):
            a = rs_[j, :]
            bp = plsc.load_gather(rd_, [jnp.zeros((16,), jnp.int32) + j, perm])
            e = a + bp
            e = jnp.where(e >= 0, e, 0.2 * e)
            ex = jnp.exp(e - mvv)
            exs[j, :] = ex
            plsc.store_scatter(
                exT, [lax.iota(jnp.int32, 16),
                      jnp.zeros((16,), jnp.int32) + (c * _B + j)],
                ex, mask=lax.iota(jnp.int32, 16) < 8)

        pltpu.sync_copy(exs, acc.at[dstst.at[c]], add=True)

    issue(0, rs0, rd0, sem0)

    def pair(i, carry):
        c0 = 2 * i
        issue(c0 + 1, rs1, rd1, sem1)
        process(c0, rs0, rd0, sem0)
        issue(c0 + 2, rs0, rd0, sem0)
        process(c0 + 1, rs1, rd1, sem1)
        return carry

    lax.fori_loop(0, (_NCHUNK - 1) // 2, pair, 0)
    process(_NCHUNK - 1, rs0, rd0, sem0)
    plsc.subcore_barrier()

    pltpu.sync_copy(exT, ext_h.at[wid])
    for i in range(_RPT // _B):
        sl = pl.ds(sid * _RPT + i * _B, _B)
        pltpu.sync_copy(acc.at[sl], rs0)
        pltpu.sync_copy(rs0, s1p_h.at[cid, sl])


def _s1(src, dst, asd, m1vec):
    zr = jnp.zeros((_B, 16), jnp.float32)
    src3 = src.reshape(_NTILES, _NCHUNK, _B)
    dst3 = dst.reshape(_NTILES, _NCHUNK, _B)
    f = pl.kernel(
        _s1_body,
        out_type=[
            jax.ShapeDtypeStruct((_NTILES, _H, _EPT), jnp.float32),
            jax.ShapeDtypeStruct((2, _NPAD, 16), jnp.float32),
        ],
        mesh=_mesh(),
        scratch_types=[
            pltpu.VMEM((_NCHUNK, _B), jnp.int32),
            pltpu.VMEM((_NCHUNK, _B), jnp.int32),
            pltpu.VMEM((_B, 16), jnp.float32),
            pltpu.VMEM((_B, 16), jnp.float32),
            pltpu.VMEM((_B, 16), jnp.float32),
            pltpu.VMEM((_B, 16), jnp.float32),
            pltpu.VMEM((_B, 16), jnp.float32),
            pltpu.VMEM((_H, _EPT), jnp.float32),
            pltpu.VMEM((16,), jnp.float32),
            pltpu.VMEM_SHARED((_NPAD, 16), jnp.float32),
            pltpu.SemaphoreType.DMA,
            pltpu.SemaphoreType.DMA,
        ],
        compiler_params=_SC_PARAMS,
    )
    return f(src3, dst3, asd, m1vec, zr)


def _s2_body(src3_h, dst3_h, h1r_h, ext4_h, zr_h,
             out_h,
             srcst, dstst, gidx0, gidx1, exb0, exb1, rows0, rows1,
             acc, sem0, sem1):
    cid = lax.axis_index("c")
    sid = lax.axis_index("s")
    wid = sid * 2 + cid

    pltpu.sync_copy(src3_h.at[wid], srcst)
    pltpu.sync_copy(dst3_h.at[wid], dstst)

    for h in range(_H):
        pltpu.sync_copy(zr_h, rows0)
        for i in range(_RPT // _B):
            pltpu.sync_copy(rows0, acc.at[pl.ds(sid * _RPT + i * _B, _B)])
        plsc.subcore_barrier()

        def issue(c, exb, gidx, rows, sem):
            pltpu.sync_copy(ext4_h.at[wid, h, c], exb)
            for k in range(_B // 16):
                sl = pl.ds(k * 16, 16)
                gidx[sl] = srcst[c, sl] * 8 + h
            pltpu.async_copy(h1r_h.at[gidx], rows, sem)

        def process(c, exb, gidx, rows, sem):
            pltpu.make_async_copy(h1r_h.at[gidx], rows, sem).wait()

            @plsc.parallel_loop(0, _B, unroll=4)
            def jbody(j):
                w = plsc.load_gather(exb, [jnp.zeros((16,), jnp.int32) + j])
                for k in range(_D // 16):
                    sl = pl.ds(k * 16, 16)
                    rows[j, sl] = rows[j, sl] * w

            pltpu.sync_copy(rows, acc.at[dstst.at[c]], add=True)

        issue(0, exb0, gidx0, rows0, sem0)

        def pair(i, carry):
            c0 = 2 * i
            issue(c0 + 1, exb1, gidx1, rows1, sem1)
            process(c0, exb0, gidx0, rows0, sem0)
            issue(c0 + 2, exb0, gidx0, rows0, sem0)
            process(c0 + 1, exb1, gidx1, rows1, sem1)
            return carry

        lax.fori_loop(0, (_NCHUNK - 1) // 2, pair, 0)
        process(_NCHUNK - 1, exb0, gidx0, rows0, sem0)
        plsc.subcore_barrier()

        for i in range(_RPT // _B):
            sl = pl.ds(sid * _RPT + i * _B, _B)
            pltpu.sync_copy(acc.at[sl], rows0)
            pltpu.sync_copy(rows0, out_h.at[cid, h, sl])
        plsc.subcore_barrier()


def _s2(src, dst, h1r, ext):
    zr = jnp.zeros((_B, _D), jnp.float32)
    src3 = src.reshape(_NTILES, _NCHUNK, _B)
    dst3 = dst.reshape(_NTILES, _NCHUNK, _B)
    ext4 = ext.reshape(_NTILES, _H, _NCHUNK, _B)
    f = pl.kernel(
        _s2_body,
        out_type=jax.ShapeDtypeStruct((2, _H, _NPAD, _D), jnp.float32),
        mesh=_mesh(),
        scratch_types=[
            pltpu.VMEM((_NCHUNK, _B), jnp.int32),
            pltpu.VMEM((_NCHUNK, _B), jnp.int32),
            pltpu.VMEM((_B,), jnp.int32),
            pltpu.VMEM((_B,), jnp.int32),
            pltpu.VMEM((_B,), jnp.float32),
            pltpu.VMEM((_B,), jnp.float32),
            pltpu.VMEM((_B, _D), jnp.float32),
            pltpu.VMEM((_B, _D), jnp.float32),
            pltpu.VMEM_SHARED((_NPAD, _D), jnp.float32),
            pltpu.SemaphoreType.DMA,
            pltpu.SemaphoreType.DMA,
        ],
        compiler_params=_SC_PARAMS,
    )
    return f(src3, dst3, h1r, ext4, zr)


def _k3a_body(p_ref, h1_ref, aux_ref, s_ref, m1_ref, b1_ref,
              o_ref, sums_ref, sq_ref):
    aux = aux_ref[...]
    a_s = aux[:, :_H]
    a_d = aux[:, _H:2 * _H]
    el = a_s + a_d
    el = jnp.where(el >= 0, el, 0.2 * el)
    exw = jnp.exp(el - m1_ref[...][0:1, :])
    s_tot = s_ref[0][:, :_H] + s_ref[1][:, :_H] + exw
    inv = 1.0 / s_tot
    parts = []
    for h in range(_H):
        num = (p_ref[0, h] + p_ref[1, h]
               + exw[:, h:h + 1] * h1_ref[:, h * _HID:(h + 1) * _HID])
        parts.append(num * inv[:, h:h + 1])
    hout = jnp.concatenate(parts, axis=1) + b1_ref[...][0:1, :]
    o_ref[...] = hout
    cs = jnp.broadcast_to(jnp.sum(hout, axis=0)[None, :], (8, _H * _HID))
    css = jnp.broadcast_to(jnp.sum(hout * hout, axis=0)[None, :],
                           (8, _H * _HID))

    @pl.when(pl.program_id(0) == 0)
    def _():
        sums_ref[...] = cs
        sq_ref[...] = css

    @pl.when(pl.program_id(0) > 0)
    def _():
        sums_ref[...] += cs
        sq_ref[...] += css


def _k3a(out1p, h1, aux, s1p, m1b, bias1b, blk=1000):
    return pl.pallas_call(
        _k3a_body,
        grid=(_N // blk,),
        in_specs=[
            pl.BlockSpec((2, _H, blk, _D), lambda i: (0, 0, i, 0)),
            pl.BlockSpec((blk, _H * _HID), lambda i: (i, 0)),
            pl.BlockSpec((blk, _D), lambda i: (i, 0)),
            pl.BlockSpec((2, blk, 16), lambda i: (0, i, 0)),
            pl.BlockSpec((8, _H), lambda i: (0, 0)),
            pl.BlockSpec((8, _H * _HID), lambda i: (0, 0)),
        ],
        out_specs=[
            pl.BlockSpec((blk, _H * _HID), lambda i: (i, 0)),
            pl.BlockSpec((8, _H * _HID), lambda i: (0, 0)),
            pl.BlockSpec((8, _H * _HID), lambda i: (0, 0)),
        ],
        out_shape=[
            jax.ShapeDtypeStruct((_N, _H * _HID), jnp.float32),
            jax.ShapeDtypeStruct((8, _H * _HID), jnp.float32),
            jax.ShapeDtypeStruct((8, _H * _HID), jnp.float32),
        ],
    )(out1p, h1, aux, s1p, m1b, bias1b)


def _k3b_body(h_ref, sc_ref, sh_ref, w_ref, o_ref):
    hb = h_ref[...] * sc_ref[...][0:1, :] + sh_ref[...][0:1, :]
    he = jnp.where(hb > 0, hb, jnp.exp(hb) - 1.0)
    o_ref[...] = jnp.dot(he, w_ref[...], preferred_element_type=jnp.float32)


def _k3b(h, scale8, shift8, w2e, blk=1000):
    k = h.shape[1]
    n = w2e.shape[1]
    return pl.pallas_call(
        _k3b_body,
        grid=(_N // blk,),
        in_specs=[
            pl.BlockSpec((blk, k), lambda i: (i, 0)),
            pl.BlockSpec((8, k), lambda i: (0, 0)),
            pl.BlockSpec((8, k), lambda i: (0, 0)),
            pl.BlockSpec((k, n), lambda i: (0, 0)),
        ],
        out_specs=pl.BlockSpec((blk, n), lambda i: (i, 0)),
        out_shape=jax.ShapeDtypeStruct((_N, n), jnp.float32),
    )(h, scale8, shift8, w2e)


def _s3_body(src3_h, dst3_h, z2e_h, ad2_h, m2_h, zr_h,
             out_h,
             ad2v, srcb0, srcb1, dstb0, dstb1, adb, rows0, rows1, mv, acc,
             sem0, sem1):
    cid = lax.axis_index("c")
    sid = lax.axis_index("s")
    wid = sid * 2 + cid

    pltpu.sync_copy(ad2_h, ad2v)
    pltpu.sync_copy(m2_h, mv)
    pltpu.sync_copy(zr_h, rows0)
    for i in range(_RPT // _B):
        pltpu.sync_copy(rows0, acc.at[pl.ds(sid * _RPT + i * _B, _B)])
    plsc.subcore_barrier()

    def issue(c, srcb_, dstb_, rows_, sem_):
        pltpu.sync_copy(src3_h.at[wid, c], srcb_)
        pltpu.sync_copy(dst3_h.at[wid, c], dstb_.at[0])
        pltpu.async_copy(z2e_h.at[srcb_], rows_, sem_)

    def process(c, srcb_, dstb_, rows_, sem_):
        pltpu.make_async_copy(z2e_h.at[srcb_], rows_, sem_).wait()
        mvv = mv[...]
        for k in range(_B // 16):
            sl = pl.ds(k * 16, 16)
            adb[sl] = plsc.load_gather(ad2v, [dstb_[0, sl]])

        @plsc.parallel_loop(0, _B, unroll=4)
        def jbody(j):
            jv = jnp.zeros((16,), jnp.int32) + j
            wa = plsc.load_gather(rows_, [jv, jnp.full((16,), 129, jnp.int32)])
            wd = plsc.load_gather(adb, [jv])
            e = wa + wd
            e = jnp.where(e >= 0, e, 0.2 * e)
            w = jnp.exp(e - mvv)
            for k in range(_W2COLS // 16):
                sl = pl.ds(k * 16, 16)
                rows_[j, sl] = rows_[j, sl] * w

        pltpu.sync_copy(rows_, acc.at[dstb_.at[0]], add=True)

    issue(0, srcb0, dstb0, rows0, sem0)

    def pair(i, carry):
        c0 = 2 * i
        issue(c0 + 1, srcb1, dstb1, rows1, sem1)
        process(c0, srcb0, dstb0, rows0, sem0)
        issue(c0 + 2, srcb0, dstb0, rows0, sem0)
        process(c0 + 1, srcb1, dstb1, rows1, sem1)
        return carry

    lax.fori_loop(0, (_NCHUNK - 1) // 2, pair, 0)
    process(_NCHUNK - 1, srcb0, dstb0, rows0, sem0)
    plsc.subcore_barrier()

    for i in range(_RPT // _B):
        sl = pl.ds(sid * _RPT + i * _B, _B)
        pltpu.sync_copy(acc.at[sl], rows0)
        pltpu.sync_copy(rows0, out_h.at[cid, sl])


def _s3(src, dst, z2e, ad2, m2vec):
    zr = jnp.zeros((_B, _W2COLS), jnp.float32)
    src3 = src.reshape(_NTILES, _NCHUNK, _B)
    dst3 = dst.reshape(_NTILES, _NCHUNK, _B)
    f = pl.kernel(
        _s3_body,
        out_type=jax.ShapeDtypeStruct((2, _NPAD, _W2COLS), jnp.float32),
        mesh=_mesh(),
        scratch_types=[
            pltpu.VMEM((_N,), jnp.float32),
            pltpu.VMEM((_B,), jnp.int32),
            pltpu.VMEM((_B,), jnp.int32),
            pltpu.VMEM((1, _B), jnp.int32),
            pltpu.VMEM((1, _B), jnp.int32),
            pltpu.VMEM((_B,), jnp.float32),
            pltpu.VMEM((_B, _W2COLS), jnp.float32),
            pltpu.VMEM((_B, _W2COLS), jnp.float32),
            pltpu.VMEM((16,), jnp.float32),
            pltpu.VMEM_SHARED((_NPAD, _W2COLS), jnp.float32),
            pltpu.SemaphoreType.DMA,
            pltpu.SemaphoreType.DMA,
        ],
        compiler_params=_SC_PARAMS,
    )
    return f(src3, dst3, z2e, ad2, m2vec, zr)


def _k4_body(p_ref, z_ref, aux_ref, m2_ref, b2_ref, o_ref):
    p0 = p_ref[0]
    p1 = p_ref[1]
    z = z_ref[...]
    a = aux_ref[...]
    m2 = m2_ref[...][0:1, 0:1]
    el = a[:, 0:1] + a[:, 1:2]
    el = jnp.where(el >= 0, el, 0.2 * el)
    exw = jnp.exp(el - m2)
    num = p0[:, :_D] + p1[:, :_D] + exw * z[:, :_D]
    den = p0[:, _D:_D + 1] + p1[:, _D:_D + 1] + exw
    o_ref[...] = num / den + b2_ref[...][0:1, :]


def _k4(out2p, z2e, aux2, m2arr, bias2b, blk=1000):
    return pl.pallas_call(
        _k4_body,
        grid=(_N // blk,),
        in_specs=[
            pl.BlockSpec((2, blk, _W2COLS), lambda i: (0, i, 0)),
            pl.BlockSpec((blk, _W2COLS), lambda i: (i, 0)),
            pl.BlockSpec((blk, 16), lambda i: (i, 0)),
            pl.BlockSpec((8, 128), lambda i: (0, 0)),
            pl.BlockSpec((8, 128), lambda i: (0, 0)),
        ],
        out_specs=pl.BlockSpec((blk, _D), lambda i: (i, 0)),
        out_shape=jax.ShapeDtypeStruct((_N, _D), jnp.float32),
    )(out2p, z2e, aux2, m2arr, bias2b)


def kernel(x, edge_index, W1, att_src1, att_dst1, bias1, gamma, beta, W2,
           att_src2, att_dst2, bias2):
    src, dst = edge_index[0], edge_index[1]

    W1r = W1.reshape(_D, _H, _HID)
    As1 = jnp.einsum('dhc,hc->dh', W1r, att_src1)
    Ad1 = jnp.einsum('dhc,hc->dh', W1r, att_dst1)
    pad1 = jnp.zeros((_D, 128 - 2 * _H), jnp.float32)
    Waux1 = jnp.concatenate([As1, Ad1, pad1], axis=1)

    h1, aux = _matmul2(x, W1, Waux1)
    asd = aux[:, :16]
    m1 = jnp.max(aux[:, :_H], axis=0) + jnp.max(aux[:, _H:2 * _H], axis=0)
    m1vec = jnp.concatenate([m1, jnp.full((8,), 1e30, jnp.float32)])
    m1b = jnp.broadcast_to(m1[None, :], (8, _H))

    ext, s1p = _s1(src, dst, asd, m1vec)
    h1r = h1.reshape(_N * _H, _HID)
    out1p = _s2(src, dst, h1r, ext)

    bias1b = jnp.broadcast_to(bias1[None, :], (8, _H * _HID))
    h, sums, sq = _k3a(out1p[:, :, :_N, :], h1, aux, s1p[:, :_N, :],
                       m1b, bias1b)

    mean = sums[0] / _N
    var = sq[0] / _N - mean * mean
    scale = gamma / jnp.sqrt(var + 1e-5)
    shift = beta - mean * scale
    scale8 = jnp.broadcast_to(scale[None, :], (8, _H * _HID))
    shift8 = jnp.broadcast_to(shift[None, :], (8, _H * _HID))

    W2r = W2.reshape(_H * _HID, 1, _D)
    As2 = jnp.einsum('dhc,hc->dh', W2r, att_src2)
    Ad2 = jnp.einsum('dhc,hc->dh', W2r, att_dst2)
    pad2 = jnp.zeros((_H * _HID, 14), jnp.float32)
    W2e = jnp.concatenate([W2, As2, Ad2, pad2], axis=1)

    h2e = _k3b(h, scale8, shift8, W2e)
    z2 = h2e[:, :_D]
    as2 = h2e[:, _D]
    ad2 = h2e[:, _D + 1]
    aux2 = h2e[:, _D:_D + 16]

    m2 = jnp.max(as2) + jnp.max(ad2)
    m2vec = jnp.full((16,), m2, jnp.float32)
    m2arr = jnp.full((8, 128), m2, jnp.float32)

    extra = jnp.concatenate(
        [jnp.ones((_N, 1), jnp.float32), as2[:, None],
         jnp.zeros((_N, _W2COLS - _D - 2), jnp.float32)], axis=1)
    z2e = jnp.concatenate([z2, extra], axis=1)

    out2p = _s3(src, dst, z2e, ad2, m2vec)[:, :_N, :]

    bias2b = jnp.broadcast_to(bias2[None, :], (8, 128))
    return _k4(out2p, z2e, aux2, m2arr, bias2b)


# speedup vs baseline: 26.8428x; 1.0400x over previous
import functools

import jax
import jax.numpy as jnp
from jax import lax
from jax.experimental import pallas as pl
from jax.experimental.pallas import tpu as pltpu
from jax.experimental.pallas import tpu_sc as plsc

_N = 10000
_E = 320000
_D = 128
_H = 8
_HID = 128

_NTILES = 32
_EPT = _E // _NTILES
_B = 80
_NCHUNK = _EPT // _B
_NPAD = 10240
_RPT = _NPAD // 16
_W2COLS = 144

_SC_PARAMS = pltpu.CompilerParams(
    use_tc_tiling_on_sc=False, needs_layout_passes=False)


def _mesh():
    return plsc.VectorSubcoreMesh(core_axis_name="c", subcore_axis_name="s")


def _mm_body(x_ref, w_ref, o_ref):
    o_ref[...] = jnp.dot(x_ref[...], w_ref[...],
                         preferred_element_type=jnp.float32)


def _matmul(x, w, blk_m=2000):
    m, k = x.shape
    _, n = w.shape
    return pl.pallas_call(
        _mm_body,
        grid=(m // blk_m,),
        in_specs=[
            pl.BlockSpec((blk_m, k), lambda i: (i, 0)),
            pl.BlockSpec((k, n), lambda i: (0, 0)),
        ],
        out_specs=pl.BlockSpec((blk_m, n), lambda i: (i, 0)),
        out_shape=jax.ShapeDtypeStruct((m, n), jnp.float32),
    )(x, w)


def _s1_body(src3_h, dst3_h, asd_h, m_h, zr_h,
             ext_h, s1p_h,
             srcst, dstst, rs0, rd0, rs1, rd1, exs, exT, mv, acc,
             sem0, sem1):
    cid = lax.axis_index("c")
    sid = lax.axis_index("s")
    wid = sid * 2 + cid
    perm = (lax.iota(jnp.int32, 16) % 8) + 8

    pltpu.sync_copy(src3_h.at[wid], srcst)
    pltpu.sync_copy(dst3_h.at[wid], dstst)
    pltpu.sync_copy(m_h, mv)
    pltpu.sync_copy(zr_h, exs)
    for i in range(_RPT // _B):
        pltpu.sync_copy(exs, acc.at[pl.ds(sid * _RPT + i * _B, _B)])
    plsc.subcore_barrier()

    def issue(c, rs_, rd_, sem_):
        pltpu.async_copy(asd_h.at[srcst.at[c]], rs_, sem_)
        pltpu.async_copy(asd_h.at[dstst.at[c]], rd_, sem_)

    def process(c, rs_, rd_, sem_):
        pltpu.make_async_copy(asd_h.at[srcst.at[c]], rs_, sem_).wait()
        pltpu.make_async_copy(asd_h.at[dstst.at[c]], rd_, sem_).wait()
        mvv = mv[...]

        @plsc.parallel_loop(0, _B, unroll=4)
        def jbody(j):
            a = rs_[j, :]
            bp = plsc.load_gather(rd_, [jnp.zeros((16,), jnp.int32) + j, perm])
            e = a + bp
            e = jnp.where(e >= 0, e, 0.2 * e)
            ex = jnp.exp(e - mvv)
            exs[j, :] = ex
            plsc.store_scatter(
                exT, [lax.iota(jnp.int32, 16),
                      jnp.zeros((16,), jnp.int32) + (c * _B + j)],
                ex, mask=lax.iota(jnp.int32, 16) < 8)

        pltpu.sync_copy(exs, acc.at[dstst.at[c]], add=True)

    issue(0, rs0, rd0, sem0)

    def pair(i, carry):
        c0 = 2 * i
        issue(c0 + 1, rs1, rd1, sem1)
        process(c0, rs0, rd0, sem0)
        issue(c0 + 2, rs0, rd0, sem0)
        process(c0 + 1, rs1, rd1, sem1)
        return carry

    lax.fori_loop(0, (_NCHUNK - 1) // 2, pair, 0)
    process(_NCHUNK - 1, rs0, rd0, sem0)
    plsc.subcore_barrier()

    pltpu.sync_copy(exT, ext_h.at[wid])
    sl = pl.ds(sid * _RPT, _RPT)
    pltpu.sync_copy(acc.at[sl], s1p_h.at[cid, sl])


def _s1(src, dst, asd, m1vec):
    zr = jnp.zeros((_B, 16), jnp.float32)
    src3 = src.reshape(_NTILES, _NCHUNK, _B)
    dst3 = dst.reshape(_NTILES, _NCHUNK, _B)
    f = pl.kernel(
        _s1_body,
        out_type=[
            jax.ShapeDtypeStruct((_NTILES, _H, _EPT), jnp.float32),
            jax.ShapeDtypeStruct((2, _NPAD, 16), jnp.float32),
        ],
        mesh=_mesh(),
        scratch_types=[
            pltpu.VMEM((_NCHUNK, _B), jnp.int32),
            pltpu.VMEM((_NCHUNK, _B), jnp.int32),
            pltpu.VMEM((_B, 16), jnp.float32),
            pltpu.VMEM((_B, 16), jnp.float32),
            pltpu.VMEM((_B, 16), jnp.float32),
            pltpu.VMEM((_B, 16), jnp.float32),
            pltpu.VMEM((_B, 16), jnp.float32),
            pltpu.VMEM((_H, _EPT), jnp.float32),
            pltpu.VMEM((16,), jnp.float32),
            pltpu.VMEM_SHARED((_NPAD, 16), jnp.float32),
            pltpu.SemaphoreType.DMA,
            pltpu.SemaphoreType.DMA,
        ],
        compiler_params=_SC_PARAMS,
    )
    return f(src3, dst3, asd, m1vec, zr)


def _s2_body(src3_h, dst3_h, h1r_h, ext4_h, zr_h,
             out_h,
             srcst, dstst, gidx0, gidx1, exb0, exb1, rows0, rows1,
             acc, sem0, sem1):
    cid = lax.axis_index("c")
    sid = lax.axis_index("s")
    wid = sid * 2 + cid

    pltpu.sync_copy(src3_h.at[wid], srcst)
    pltpu.sync_copy(dst3_h.at[wid], dstst)

    for h in range(_H):
        pltpu.sync_copy(zr_h, rows0)
        for i in range(_RPT // _B):
            pltpu.sync_copy(rows0, acc.at[pl.ds(sid * _RPT + i * _B, _B)])
        plsc.subcore_barrier()

        def issue(c, exb, gidx, rows, sem):
            pltpu.sync_copy(ext4_h.at[wid, h, c], exb)
            for k in range(_B // 16):
                sl = pl.ds(k * 16, 16)
                gidx[sl] = srcst[c, sl] * 8 + h
            pltpu.async_copy(h1r_h.at[gidx], rows, sem)

        def process(c, exb, gidx, rows, sem):
            pltpu.make_async_copy(h1r_h.at[gidx], rows, sem).wait()

            @plsc.parallel_loop(0, _B, unroll=4)
            def jbody(j):
                w = plsc.load_gather(exb, [jnp.zeros((16,), jnp.int32) + j])
                for k in range(_D // 16):
                    sl = pl.ds(k * 16, 16)
                    rows[j, sl] = rows[j, sl] * w

            pltpu.sync_copy(rows, acc.at[dstst.at[c]], add=True)

        issue(0, exb0, gidx0, rows0, sem0)

        def pair(i, carry):
            c0 = 2 * i
            issue(c0 + 1, exb1, gidx1, rows1, sem1)
            process(c0, exb0, gidx0, rows0, sem0)
            issue(c0 + 2, exb0, gidx0, rows0, sem0)
            process(c0 + 1, exb1, gidx1, rows1, sem1)
            return carry

        lax.fori_loop(0, (_NCHUNK - 1) // 2, pair, 0)
        process(_NCHUNK - 1, exb0, gidx0, rows0, sem0)
        plsc.subcore_barrier()

        sl = pl.ds(sid * _RPT, _RPT)
        pltpu.sync_copy(acc.at[sl], out_h.at[cid, h, sl])
        plsc.subcore_barrier()


def _s2(src, dst, h1r, ext):
    zr = jnp.zeros((_B, _D), jnp.float32)
    src3 = src.reshape(_NTILES, _NCHUNK, _B)
    dst3 = dst.reshape(_NTILES, _NCHUNK, _B)
    ext4 = ext.reshape(_NTILES, _H, _NCHUNK, _B)
    f = pl.kernel(
        _s2_body,
        out_type=jax.ShapeDtypeStruct((2, _H, _NPAD, _D), jnp.float32),
        mesh=_mesh(),
        scratch_types=[
            pltpu.VMEM((_NCHUNK, _B), jnp.int32),
            pltpu.VMEM((_NCHUNK, _B), jnp.int32),
            pltpu.VMEM((_B,), jnp.int32),
            pltpu.VMEM((_B,), jnp.int32),
            pltpu.VMEM((_B,), jnp.float32),
            pltpu.VMEM((_B,), jnp.float32),
            pltpu.VMEM((_B, _D), jnp.float32),
            pltpu.VMEM((_B, _D), jnp.float32),
            pltpu.VMEM_SHARED((_NPAD, _D), jnp.float32),
            pltpu.SemaphoreType.DMA,
            pltpu.SemaphoreType.DMA,
        ],
        compiler_params=_SC_PARAMS,
    )
    return f(src3, dst3, h1r, ext4, zr)


def _k3a_body(p_ref, h1_ref, aux_ref, s_ref, m1_ref, b1_ref,
              o_ref, sums_ref, sq_ref):
    aux = aux_ref[...]
    a_s = aux[:, :_H]
    a_d = aux[:, _H:2 * _H]
    el = a_s + a_d
    el = jnp.where(el >= 0, el, 0.2 * el)
    exw = jnp.exp(el - m1_ref[...][0:1, :])
    s_tot = s_ref[0][:, :_H] + s_ref[1][:, :_H] + exw
    inv = 1.0 / s_tot
    parts = []
    for h in range(_H):
        num = (p_ref[0, h] + p_ref[1, h]
               + exw[:, h:h + 1] * h1_ref[:, h * _HID:(h + 1) * _HID])
        parts.append(num * inv[:, h:h + 1])
    hout = jnp.concatenate(parts, axis=1) + b1_ref[...][0:1, :]
    o_ref[...] = hout
    cs = jnp.broadcast_to(jnp.sum(hout, axis=0)[None, :], (8, _H * _HID))
    css = jnp.broadcast_to(jnp.sum(hout * hout, axis=0)[None, :],
                           (8, _H * _HID))

    @pl.when(pl.program_id(0) == 0)
    def _():
        sums_ref[...] = cs
        sq_ref[...] = css

    @pl.when(pl.program_id(0) > 0)
    def _():
        sums_ref[...] += cs
        sq_ref[...] += css


def _k3a(out1p, h1, aux, s1p, m1b, bias1b, blk=1000):
    return pl.pallas_call(
        _k3a_body,
        grid=(_N // blk,),
        in_specs=[
            pl.BlockSpec((2, _H, blk, _D), lambda i: (0, 0, i, 0)),
            pl.BlockSpec((blk, _H * _HID), lambda i: (i, 0)),
            pl.BlockSpec((blk, _D), lambda i: (i, 0)),
            pl.BlockSpec((2, blk, 16), lambda i: (0, i, 0)),
            pl.BlockSpec((8, _H), lambda i: (0, 0)),
            pl.BlockSpec((8, _H * _HID), lambda i: (0, 0)),
        ],
        out_specs=[
            pl.BlockSpec((blk, _H * _HID), lambda i: (i, 0)),
            pl.BlockSpec((8, _H * _HID), lambda i: (0, 0)),
            pl.BlockSpec((8, _H * _HID), lambda i: (0, 0)),
        ],
        out_shape=[
            jax.ShapeDtypeStruct((_N, _H * _HID), jnp.float32),
            jax.ShapeDtypeStruct((8, _H * _HID), jnp.float32),
            jax.ShapeDtypeStruct((8, _H * _HID), jnp.float32),
        ],
    )(out1p, h1, aux, s1p, m1b, bias1b)


def _k3b_body(h_ref, sc_ref, sh_ref, w_ref, ec_ref, o_ref):
    hb = h_ref[...] * sc_ref[...][0:1, :] + sh_ref[...][0:1, :]
    he = jnp.where(hb > 0, hb, jnp.exp(hb) - 1.0)
    mm = jnp.dot(he, w_ref[...], preferred_element_type=jnp.float32)
    o_ref[...] = mm + ec_ref[...][0:1, :]


def _k3b(h, scale8, shift8, w2e, ec, blk=1000):
    k = h.shape[1]
    n = w2e.shape[1]
    return pl.pallas_call(
        _k3b_body,
        grid=(_N // blk,),
        in_specs=[
            pl.BlockSpec((blk, k), lambda i: (i, 0)),
            pl.BlockSpec((8, k), lambda i: (0, 0)),
            pl.BlockSpec((8, k), lambda i: (0, 0)),
            pl.BlockSpec((k, n), lambda i: (0, 0)),
            pl.BlockSpec((8, n), lambda i: (0, 0)),
        ],
        out_specs=pl.BlockSpec((blk, n), lambda i: (i, 0)),
        out_shape=jax.ShapeDtypeStruct((_N, n), jnp.float32),
    )(h, scale8, shift8, w2e, ec)


def _s3_body(src3_h, dst3_h, z2e_h, ad2_h, m2_h, zr_h,
             out_h,
             ad2v, srcb0, srcb1, dstb0, dstb1, adb, rows0, rows1, mv, acc,
             sem0, sem1):
    cid = lax.axis_index("c")
    sid = lax.axis_index("s")
    wid = sid * 2 + cid

    pltpu.sync_copy(ad2_h, ad2v)
    pltpu.sync_copy(m2_h, mv)
    pltpu.sync_copy(zr_h, rows0)
    for i in range(_RPT // _B):
        pltpu.sync_copy(rows0, acc.at[pl.ds(sid * _RPT + i * _B, _B)])
    plsc.subcore_barrier()

    def issue(c, srcb_, dstb_, rows_, sem_):
        pltpu.sync_copy(src3_h.at[wid, c], srcb_)
        pltpu.sync_copy(dst3_h.at[wid, c], dstb_.at[0])
        pltpu.async_copy(z2e_h.at[srcb_], rows_, sem_)

    def process(c, srcb_, dstb_, rows_, sem_):
        pltpu.make_async_copy(z2e_h.at[srcb_], rows_, sem_).wait()
        mvv = mv[...]
        for k in range(_B // 16):
            sl = pl.ds(k * 16, 16)
            adb[sl] = plsc.load_gather(ad2v, [dstb_[0, sl]])

        @plsc.parallel_loop(0, _B, unroll=4)
        def jbody(j):
            jv = jnp.zeros((16,), jnp.int32) + j
            wa = plsc.load_gather(rows_, [jv, jnp.full((16,), 129, jnp.int32)])
            wd = plsc.load_gather(adb, [jv])
            e = wa + wd
            e = jnp.where(e >= 0, e, 0.2 * e)
            w = jnp.exp(e - mvv)
            for k in range(_W2COLS // 16):
                sl = pl.ds(k * 16, 16)
                rows_[j, sl] = rows_[j, sl] * w

        pltpu.sync_copy(rows_, acc.at[dstb_.at[0]], add=True)

    issue(0, srcb0, dstb0, rows0, sem0)

    def pair(i, carry):
        c0 = 2 * i
        issue(c0 + 1, srcb1, dstb1, rows1, sem1)
        process(c0, srcb0, dstb0, rows0, sem0)
        issue(c0 + 2, srcb0, dstb0, rows0, sem0)
        process(c0 + 1, srcb1, dstb1, rows1, sem1)
        return carry

    lax.fori_loop(0, (_NCHUNK - 1) // 2, pair, 0)
    process(_NCHUNK - 1, srcb0, dstb0, rows0, sem0)
    plsc.subcore_barrier()

    sl = pl.ds(sid * _RPT, _RPT)
    pltpu.sync_copy(acc.at[sl], out_h.at[cid, sl])


def _s3(src, dst, z2e, ad2, m2vec):
    zr = jnp.zeros((_B, _W2COLS), jnp.float32)
    src3 = src.reshape(_NTILES, _NCHUNK, _B)
    dst3 = dst.reshape(_NTILES, _NCHUNK, _B)
    f = pl.kernel(
        _s3_body,
        out_type=jax.ShapeDtypeStruct((2, _NPAD, _W2COLS), jnp.float32),
        mesh=_mesh(),
        scratch_types=[
            pltpu.VMEM((_N,), jnp.float32),
            pltpu.VMEM((_B,), jnp.int32),
            pltpu.VMEM((_B,), jnp.int32),
            pltpu.VMEM((1, _B), jnp.int32),
            pltpu.VMEM((1, _B), jnp.int32),
            pltpu.VMEM((_B,), jnp.float32),
            pltpu.VMEM((_B, _W2COLS), jnp.float32),
            pltpu.VMEM((_B, _W2COLS), jnp.float32),
            pltpu.VMEM((16,), jnp.float32),
            pltpu.VMEM_SHARED((_NPAD, _W2COLS), jnp.float32),
            pltpu.SemaphoreType.DMA,
            pltpu.SemaphoreType.DMA,
        ],
        compiler_params=_SC_PARAMS,
    )
    return f(src3, dst3, z2e, ad2, m2vec, zr)


def _k4_body(p_ref, z_ref, m2_ref, b2_ref, o_ref):
    p0 = p_ref[0]
    p1 = p_ref[1]
    z = z_ref[...]
    m2 = m2_ref[...][0:1, 0:1]
    el = z[:, 129:130] + z[:, 130:131]
    el = jnp.where(el >= 0, el, 0.2 * el)
    exw = jnp.exp(el - m2)
    num = p0[:, :_D] + p1[:, :_D] + exw * z[:, :_D]
    den = p0[:, _D:_D + 1] + p1[:, _D:_D + 1] + exw
    o_ref[...] = num / den + b2_ref[...][0:1, :]


def _k4(out2p, z2e, m2arr, bias2b, blk=1000):
    return pl.pallas_call(
        _k4_body,
        grid=(_N // blk,),
        in_specs=[
            pl.BlockSpec((2, blk, _W2COLS), lambda i: (0, i, 0)),
            pl.BlockSpec((blk, _W2COLS), lambda i: (i, 0)),
            pl.BlockSpec((8, 128), lambda i: (0, 0)),
            pl.BlockSpec((8, 128), lambda i: (0, 0)),
        ],
        out_specs=pl.BlockSpec((blk, _D), lambda i: (i, 0)),
        out_shape=jax.ShapeDtypeStruct((_N, _D), jnp.float32),
    )(out2p, z2e, m2arr, bias2b)


def kernel(x, edge_index, W1, att_src1, att_dst1, bias1, gamma, beta, W2,
           att_src2, att_dst2, bias2):
    src, dst = edge_index[0], edge_index[1]

    W1r = W1.reshape(_D, _H, _HID)
    As1 = jnp.einsum('dhc,hc->dh', W1r, att_src1)
    Ad1 = jnp.einsum('dhc,hc->dh', W1r, att_dst1)
    pad1 = jnp.zeros((_D, 128 - 2 * _H), jnp.float32)
    Waux1 = jnp.concatenate([As1, Ad1, pad1], axis=1)

    aux = _matmul(x, Waux1)
    asd = aux[:, :16]
    m1 = jnp.max(aux[:, :_H], axis=0) + jnp.max(aux[:, _H:2 * _H], axis=0)
    m1vec = jnp.concatenate([m1, jnp.full((8,), 1e30, jnp.float32)])
    m1b = jnp.broadcast_to(m1[None, :], (8, _H))

    ext, s1p = _s1(src, dst, asd, m1vec)
    h1 = _matmul(x, W1)
    h1r = h1.reshape(_N * _H, _HID)
    out1p = _s2(src, dst, h1r, ext)

    bias1b = jnp.broadcast_to(bias1[None, :], (8, _H * _HID))
    h, sums, sq = _k3a(out1p, h1, aux, s1p, m1b, bias1b)

    mean = sums[0] / _N
    var = sq[0] / _N - mean * mean
    scale = gamma / jnp.sqrt(var + 1e-5)
    shift = beta - mean * scale
    scale8 = jnp.broadcast_to(scale[None, :], (8, _H * _HID))
    shift8 = jnp.broadcast_to(shift[None, :], (8, _H * _HID))

    W2r = W2.reshape(_H * _HID, 1, _D)
    As2 = jnp.einsum('dhc,hc->dh', W2r, att_src2)
    Ad2 = jnp.einsum('dhc,hc->dh', W2r, att_dst2)
    pad2 = jnp.zeros((_H * _HID, _W2COLS - _D - 3), jnp.float32)
    zc = jnp.zeros((_H * _HID, 1), jnp.float32)
    W2e = jnp.concatenate([W2, zc, As2, Ad2, pad2], axis=1)
    ec = jnp.broadcast_to(
        (lax.broadcasted_iota(jnp.int32, (8, _W2COLS), 1) == _D
         ).astype(jnp.float32), (8, _W2COLS))

    z2e = _k3b(h, scale8, shift8, W2e, ec)
    ad2 = z2e[:, _D + 2]

    m2 = jnp.max(z2e[:, _D + 1]) + jnp.max(ad2)
    m2vec = jnp.full((16,), m2, jnp.float32)
    m2arr = jnp.full((8, 128), m2, jnp.float32)

    out2p = _s3(src, dst, z2e, ad2, m2vec)

    bias2b = jnp.broadcast_to(bias2[None, :], (8, 128))
    return _k4(out2p, z2e, m2arr, bias2b)
